# Initial kernel scaffold; baseline (speedup 1.0000x reference)
import jax
import jax.numpy as jnp
from jax.experimental import pallas as pl


def kernel(positions, node_attrs, edge_index, shifts, batch, ptr, atomic_energies, W_embed, mlp1_0, mlp2_0, mlp3_0, Wmix_0, Wsc_0, Wread_0, Winv_0, mlp1_1, mlp2_1, mlp3_1, Wmix_1, Wsc_1, Wread_1, Winv_1, Wdec):
    raise NotImplementedError("write your pallas kernel here")


# speedup vs baseline: 2.0948x; 2.0948x over previous
import functools

import jax
import jax.numpy as jnp
from jax import lax
from jax.experimental import pallas as pl
from jax.experimental.pallas import tpu as pltpu
from jax.experimental.pallas import tpu_sc as plsc

N = 10000
E = 160000
F = 128
Z = 4
G = 16
NB = 8
LM = 4
NE = 3
NP = 4
R_MAX = 5.0
MSG_SCALE = 1.0 / (4.0 * 16.0)

NC = 2
NS = 16
NW = NC * NS
EPW = E // NW
CH = 128
NCH = EPW // CH
TAIL = EPW - NCH * CH
FBN = 4
NPT = N // NS

NBLK = 400
NGRID = N // NBLK
EBLK = 640
EGRID = E // EBLK

_f32 = jnp.float32
_i32 = jnp.int32


def _sc_vec(pos_t, src, dst):
    mesh = plsc.VectorSubcoreMesh(core_axis_name="c", subcore_axis_name="s")

    @functools.partial(
        pl.kernel,
        out_type=jax.ShapeDtypeStruct((E, 16), _f32),
        mesh=mesh,
        scratch_types=[
            pltpu.VMEM((CH,), _i32),
            pltpu.VMEM((TAIL,), _i32),
            pltpu.VMEM((CH, 16), _f32),
            pltpu.VMEM((CH, 16), _f32),
            pltpu.SemaphoreType.DMA,
        ],
        compiler_params=pltpu.CompilerParams(use_tc_tiling_on_sc=False),
    )
    def k(pos_hbm, src_hbm, dst_hbm, out_hbm, idxv, idxv_t, ps, pd, sem):
        wid = lax.axis_index("c") * NS + lax.axis_index("s")
        base0 = wid * EPW

        def do_chunk(base, c, iv):
            pltpu.sync_copy(src_hbm.at[pl.ds(base, c)], iv)
            pltpu.async_copy(pos_hbm.at[iv], ps.at[pl.ds(0, c)], sem).wait()
            pltpu.sync_copy(dst_hbm.at[pl.ds(base, c)], iv)
            pltpu.async_copy(pos_hbm.at[iv], pd.at[pl.ds(0, c)], sem).wait()

            def body(i, _):
                ps[i] = pd[i] - ps[i]
                return 0

            lax.fori_loop(0, c, body, 0)
            pltpu.sync_copy(ps.at[pl.ds(0, c)], out_hbm.at[pl.ds(base, c)])

        def cbody(ci, _):
            do_chunk(base0 + ci * CH, CH, idxv)
            return 0

        lax.fori_loop(0, NCH, cbody, 0)
        do_chunk(base0 + NCH * CH, TAIL, idxv_t)

    return k(pos_t, src, dst)


def _tc_edge(vec, shifts, m1_0, m2_0, m3_0, m1_1, m2_1, m3_1):
    def body(vec_ref, shf_ref, a1, a2, a3, b1, b2, b3, sh_ref, w0_ref, w1_ref):
        v = vec_ref[:, 0:3] + shf_ref[...]
        l2 = jnp.sum(v * v, axis=1) + 1e-12
        length = jnp.sqrt(l2)
        u = length * (1.0 / R_MAX)
        unit = v / length[:, None]
        s3 = 3.0 ** 0.5
        sh_ref[...] = jnp.concatenate(
            [jnp.ones((EBLK, 1), _f32), s3 * unit], axis=1)
        u2 = u * u
        u4 = u2 * u2
        u5 = u4 * u
        poly = 1.0 - 21.0 * u5 + 35.0 * u5 * u - 15.0 * u5 * u2
        cut = jnp.where(u < 1.0, poly, 0.0)
        nvec = (lax.broadcasted_iota(_i32, (1, NB), 1) + 1).astype(_f32)
        b = ((2.0 / R_MAX) ** 0.5) * jnp.sin(nvec * jnp.pi * u[:, None]) \
            / length[:, None]
        ef = b * cut[:, None]
        for (m1, m2, m3, out_ref) in ((a1, a2, a3, w0_ref),
                                      (b1, b2, b3, w1_ref)):
            x = jnp.dot(ef, m1[...], preferred_element_type=_f32)
            x = jax.nn.silu(x)
            x = jax.nn.silu(jnp.dot(x, m2[...], preferred_element_type=_f32))
            we = jnp.dot(x, m3[...], preferred_element_type=_f32) * MSG_SCALE
            for fb in range(FBN):
                out_ref[fb] = we[:, fb * 32:(fb + 1) * 32]

    wspec = [
        pl.BlockSpec((NB, 64), lambda i: (0, 0)),
        pl.BlockSpec((64, 64), lambda i: (0, 0)),
        pl.BlockSpec((64, F), lambda i: (0, 0)),
    ]
    return pl.pallas_call(
        body,
        grid=(EGRID,),
        in_specs=[
            pl.BlockSpec((EBLK, 16), lambda i: (i, 0)),
            pl.BlockSpec((EBLK, 3), lambda i: (i, 0)),
        ] + wspec + wspec,
        out_specs=[
            pl.BlockSpec((EBLK, 4), lambda i: (i, 0)),
            pl.BlockSpec((FBN, EBLK, 32), lambda i: (0, i, 0)),
            pl.BlockSpec((FBN, EBLK, 32), lambda i: (0, i, 0)),
        ],
        out_shape=[
            jax.ShapeDtypeStruct((E, 4), _f32),
            jax.ShapeDtypeStruct((FBN, E, 32), _f32),
            jax.ShapeDtypeStruct((FBN, E, 32), _f32),
        ],
    )(vec, shifts, m1_0, m2_0, m3_0, m1_1, m2_1, m3_1)


def _tc_embed(node_attrs, batch3, w_embed):
    def body(attr_ref, b3_ref, we_ref, h0_ref, seg_ref):
        attrs = attr_ref[...]
        h0 = jnp.dot(attrs, we_ref[...], preferred_element_type=_f32)
        for fb in range(FBN):
            h0_ref[fb] = h0[:, fb * 32:(fb + 1) * 32]
        bt = b3_ref[0, 0]
        oh = (bt[:, None] ==
              lax.broadcasted_iota(_i32, (NBLK, G), 1)).astype(_f32)
        sa = lax.dot_general(oh, attrs, (((0,), (0,)), ((), ())),
                             preferred_element_type=_f32)

        @pl.when(pl.program_id(0) == 0)
        def _():
            seg_ref[...] = jnp.zeros_like(seg_ref)

        seg_ref[...] += sa

    return pl.pallas_call(
        body,
        grid=(NGRID,),
        in_specs=[
            pl.BlockSpec((NBLK, Z), lambda i: (i, 0)),
            pl.BlockSpec((1, 1, NBLK), lambda i: (i, 0, 0)),
            pl.BlockSpec((Z, F), lambda i: (0, 0)),
        ],
        out_specs=[
            pl.BlockSpec((FBN, NBLK, 32), lambda i: (0, i, 0)),
            pl.BlockSpec((G, Z), lambda i: (0, 0)),
        ],
        out_shape=[
            jax.ShapeDtypeStruct((FBN, N, 32), _f32),
            jax.ShapeDtypeStruct((G, Z), _f32),
        ],
    )(node_attrs, batch3, w_embed)


def _sc_msg(h_tabs, sh, we, src, dst, first_layer):
    ht = 32 if first_layer else F
    mesh = plsc.VectorSubcoreMesh(core_axis_name="c", subcore_axis_name="s")

    @functools.partial(
        pl.kernel,
        out_type=jax.ShapeDtypeStruct((NC, FBN, N, F), _f32),
        mesh=mesh,
        scratch_types=[
            pltpu.VMEM_SHARED((N, F), _f32),
            pltpu.VMEM((CH,), _i32),
            pltpu.VMEM((CH,), _i32),
            pltpu.VMEM((TAIL,), _i32),
            pltpu.VMEM((TAIL,), _i32),
            pltpu.VMEM((CH, ht), _f32),
            pltpu.VMEM((CH, F), _f32),
            pltpu.VMEM((CH, 32), _f32),
            pltpu.VMEM((CH * 4 + 16,), _f32),
            pltpu.SemaphoreType.DMA,
        ],
        compiler_params=pltpu.CompilerParams(use_tc_tiling_on_sc=False),
    )
    def k(h0_hbm, h1_hbm, h2_hbm, h3_hbm, sh_hbm, we_hbm, src_hbm, dst_hbm,
          out_hbm, agg, srcv, dstv, srcv_t, dstv_t, rows, pay, wev, shv, sem):
        cc = lax.axis_index("c")
        ss = lax.axis_index("s")
        ebase0 = (cc * NS + ss) * EPW
        r0 = ss * NPT
        htabs = (h0_hbm, h1_hbm, h2_hbm, h3_hbm)

        for fb in range(FBN):
            def zbody(r, _):
                for gg in range(F // 16):
                    pay[r, pl.ds(gg * 16, 16)] = jnp.zeros((16,), _f32)
                return 0

            lax.fori_loop(0, CH, zbody, 0)
            off = 0
            for cnt in (128, 128, 128, 128, NPT - 512):
                pltpu.sync_copy(pay.at[pl.ds(0, cnt)],
                                agg.at[pl.ds(r0 + off, cnt)])
                off += cnt
            plsc.subcore_barrier()

            def do_chunk(base, c, sv, dv, htab):
                pltpu.sync_copy(src_hbm.at[pl.ds(base, c)], sv)
                pltpu.sync_copy(dst_hbm.at[pl.ds(base, c)], dv)
                pltpu.async_copy(htab.at[sv], rows.at[pl.ds(0, c)], sem).wait()
                pltpu.sync_copy(we_hbm.at[fb, pl.ds(base, c)],
                                wev.at[pl.ds(0, c)])
                pltpu.sync_copy(sh_hbm.at[pl.ds(base * 4, c * 4)],
                                shv.at[pl.ds(0, c * 4)])

                def ebody(i, _):
                    sv = shv[pl.ds(i * 4, 16)]
                    s1 = sv[1]
                    s2 = sv[2]
                    s3 = sv[3]
                    for gg in range(2):
                        o = gg * 16
                        if first_layer:
                            proj = rows[i, pl.ds(o, 16)]
                        else:
                            proj = (rows[i, pl.ds(o, 16)]
                                    + rows[i, pl.ds(32 + o, 16)] * s1
                                    + rows[i, pl.ds(64 + o, 16)] * s2
                                    + rows[i, pl.ds(96 + o, 16)] * s3)
                        a = proj * wev[i, pl.ds(o, 16)]
                        pay[i, pl.ds(o, 16)] = a
                        pay[i, pl.ds(32 + o, 16)] = a * s1
                        pay[i, pl.ds(64 + o, 16)] = a * s2
                        pay[i, pl.ds(96 + o, 16)] = a * s3
                    return 0

                lax.fori_loop(0, c, ebody, 0)
                if c == CH:
                    pltpu.sync_copy(pay, agg.at[dv], add=True)
                else:
                    pltpu.sync_copy(pay.at[pl.ds(0, c)], agg.at[dv], add=True)

            def cbody(ci, _):
                do_chunk(ebase0 + ci * CH, CH, srcv, dstv, htabs[fb])
                return 0

            lax.fori_loop(0, NCH, cbody, 0)
            do_chunk(ebase0 + NCH * CH, TAIL, srcv_t, dstv_t, htabs[fb])
            plsc.subcore_barrier()

            off = 0
            for cnt in (128, 128, 128, 128, NPT - 512):
                for cc_s in range(NC):
                    @pl.when(cc == cc_s)
                    def _():
                        pltpu.sync_copy(
                            agg.at[pl.ds(r0 + off, cnt)],
                            out_hbm.at[cc_s, fb, pl.ds(r0 + off, cnt)])
                off += cnt
            plsc.subcore_barrier()

    return k(h_tabs[0], h_tabs[1], h_tabs[2], h_tabs[3], sh, we, src, dst)


def _tc_node(aggp, h_tab, batch3, wm, ws, first_layer):
    ht = 32 if first_layer else F

    def body(agg_ref, h_ref, b3_ref, wm_ref, ws_ref, hn_ref, seg_ref):
        a = agg_ref[0] + agg_ref[1]
        wmv = wm_ref[...]
        wsv = ws_ref[...]
        ms = []
        bsq = jnp.zeros((NBLK, F), _f32)
        for l in range(LM):
            x = jnp.concatenate(
                [a[fb, :, l * 32:(l + 1) * 32] for fb in range(FBN)], axis=1)
            m = jnp.dot(x, wmv, preferred_element_type=_f32)
            ms.append(m)
            bsq = bsq + m * m
        t = jnp.tanh(bsq)
        inv = None
        for l in range(LM):
            if first_layer:
                if l == 0:
                    hsrc = jnp.concatenate(
                        [h_ref[fb] for fb in range(FBN)], axis=1)
                    hx = jnp.dot(hsrc, wsv, preferred_element_type=_f32)
                else:
                    hx = None
            else:
                hsrc = jnp.concatenate(
                    [h_ref[fb, :, l * 32:(l + 1) * 32] for fb in range(FBN)],
                    axis=1)
                hx = jnp.dot(hsrc, wsv, preferred_element_type=_f32)
            hn = ms[l] + ms[l] * t
            if hx is not None:
                hn = hn + hx
            for fb in range(FBN):
                hn_ref[fb, :, l * 32:(l + 1) * 32] = hn[:, fb * 32:(fb + 1) * 32]
            if l == 0:
                inv = hn
        bt = b3_ref[0, 0]
        oh = (bt[:, None] ==
              lax.broadcasted_iota(_i32, (NBLK, G), 1)).astype(_f32)
        sa = lax.dot_general(oh, inv, (((0,), (0,)), ((), ())),
                             preferred_element_type=_f32)

        @pl.when(pl.program_id(0) == 0)
        def _():
            seg_ref[...] = jnp.zeros_like(seg_ref)

        seg_ref[...] += sa

    hspec = (pl.BlockSpec((FBN, NBLK, 32), lambda i: (0, i, 0)) if first_layer
             else pl.BlockSpec((FBN, NBLK, F), lambda i: (0, i, 0)))
    return pl.pallas_call(
        body,
        grid=(NGRID,),
        in_specs=[
            pl.BlockSpec((NC, FBN, NBLK, F), lambda i: (0, 0, i, 0)),
            hspec,
            pl.BlockSpec((1, 1, NBLK), lambda i: (i, 0, 0)),
            pl.BlockSpec((F, F), lambda i: (0, 0)),
            pl.BlockSpec((F, F), lambda i: (0, 0)),
        ],
        out_specs=[
            pl.BlockSpec((FBN, NBLK, F), lambda i: (0, i, 0)),
            pl.BlockSpec((G, F), lambda i: (0, 0)),
        ],
        out_shape=[
            jax.ShapeDtypeStruct((FBN, N, F), _f32),
            jax.ShapeDtypeStruct((G, F), _f32),
        ],
    )(aggp, h_tab, batch3, wm, ws)


def _tc_final(seg_attr, ae1, seg1, seg2, wr0, wi0, wr1, wi1, wdec):
    def body(sa_ref, ae_ref, s1_ref, s2_ref, wr0_ref, wi0_ref, wr1_ref,
             wi1_ref, wd_ref, out_ref):
        e0 = jnp.dot(sa_ref[...], ae_ref[...], preferred_element_type=_f32)
        s1 = s1_ref[...]
        s2 = s2_ref[...]
        en = (jnp.dot(s1, wr0_ref[...], preferred_element_type=_f32)
              + jnp.dot(s2, wr1_ref[...], preferred_element_type=_f32))
        il = (jnp.dot(s1, wi0_ref[...], preferred_element_type=_f32)
              + jnp.dot(s2, wi1_ref[...], preferred_element_type=_f32))
        out_ref[...] = e0 + en + jnp.dot(il, wd_ref[...],
                                         preferred_element_type=_f32)

    return pl.pallas_call(
        body,
        out_shape=jax.ShapeDtypeStruct((G, NE), _f32),
    )(seg_attr, ae1, seg1, seg2, wr0, wi0, wr1, wi1, wdec)


def kernel(positions, node_attrs, edge_index, shifts, batch, ptr,
           atomic_energies, W_embed, mlp1_0, mlp2_0, mlp3_0, Wmix_0, Wsc_0,
           Wread_0, Winv_0, mlp1_1, mlp2_1, mlp3_1, Wmix_1, Wsc_1, Wread_1,
           Winv_1, Wdec):
    src = edge_index[0].astype(_i32)
    dst = edge_index[1].astype(_i32)
    pos_t = jnp.zeros((N, 16), _f32).at[:, 0:3].set(positions.astype(_f32))
    batch3 = batch.astype(_i32).reshape(NGRID, 1, NBLK)

    vec = _sc_vec(pos_t, src, dst)
    sh, we0, we1 = _tc_edge(vec, shifts.astype(_f32), mlp1_0, mlp2_0, mlp3_0,
                            mlp1_1, mlp2_1, mlp3_1)
    h0_tab, seg_attr = _tc_embed(node_attrs.astype(_f32), batch3, W_embed)

    sh_flat = sh.reshape(E * 4)
    h0s = [h0_tab[fb] for fb in range(FBN)]
    aggp1 = _sc_msg(h0s, sh_flat, we0, src, dst, first_layer=True)
    h1_tab, seg1 = _tc_node(aggp1, h0_tab, batch3, Wmix_0, Wsc_0,
                            first_layer=True)

    h1s = [h1_tab[fb] for fb in range(FBN)]
    aggp2 = _sc_msg(h1s, sh_flat, we1, src, dst, first_layer=False)
    _, seg2 = _tc_node(aggp2, h1_tab, batch3, Wmix_1, Wsc_1,
                       first_layer=False)

    return _tc_final(seg_attr, atomic_energies.reshape(Z, 1).astype(_f32),
                     seg1, seg2, Wread_0, Winv_0, Wread_1, Winv_1, Wdec)


# speedup vs baseline: 2.6969x; 1.2874x over previous
import functools

import jax
import jax.numpy as jnp
from jax import lax
from jax.experimental import pallas as pl
from jax.experimental.pallas import tpu as pltpu
from jax.experimental.pallas import tpu_sc as plsc

N = 10000
E = 160000
F = 128
Z = 4
G = 16
NB = 8
LM = 4
NE = 3
NP = 4
R_MAX = 5.0
MSG_SCALE = 1.0 / (4.0 * 16.0)

NC = 2
NS = 16
NW = NC * NS
CH = 128
NCHT = E // CH
KPW = (NCHT + NW - 1) // NW
MCH = 64
NMCH = E // MCH
KM = 80
FBN = 4
NPT = N // NS

NBLK = 400
NGRID = N // NBLK
EBLK = 640
EGRID = E // EBLK

_f32 = jnp.float32
_i32 = jnp.int32


def _sc_vec(pos_t, src, dst):
    mesh = plsc.VectorSubcoreMesh(core_axis_name="c", subcore_axis_name="s")

    @functools.partial(
        pl.kernel,
        out_type=jax.ShapeDtypeStruct((E, 16), _f32),
        mesh=mesh,
        scratch_types=[
            pltpu.VMEM((2, CH), _i32),
            pltpu.VMEM((2, CH), _i32),
            pltpu.VMEM((2, CH, 16), _f32),
            pltpu.VMEM((2, CH, 16), _f32),
            pltpu.SemaphoreType.DMA,
            pltpu.SemaphoreType.DMA,
            pltpu.SemaphoreType.DMA,
            pltpu.SemaphoreType.DMA,
        ],
        compiler_params=pltpu.CompilerParams(use_tc_tiling_on_sc=False),
    )
    def k(pos_hbm, src_hbm, dst_hbm, out_hbm, sv, dv, ps, pd,
          l0, l1, g0, g1):
        wid = lax.axis_index("c") * NS + lax.axis_index("s")
        lsems = (l0, l1)
        gsems = (g0, g1)

        def cbase(c):
            return jnp.minimum(wid + c * NW, NCHT - 1) * CH

        def stage(c, b):
            base = cbase(c)
            pltpu.async_copy(src_hbm.at[pl.ds(base, CH)], sv.at[b], lsems[b])
            pltpu.async_copy(dst_hbm.at[pl.ds(base, CH)], dv.at[b], lsems[b])

        def wait_stage(c, b):
            base = cbase(c)
            pltpu.make_async_copy(src_hbm.at[pl.ds(base, CH)], sv.at[b],
                                  lsems[b]).wait()
            pltpu.make_async_copy(dst_hbm.at[pl.ds(base, CH)], dv.at[b],
                                  lsems[b]).wait()

        def start_gather(b):
            pltpu.async_copy(pos_hbm.at[sv.at[b]], ps.at[b], gsems[b])
            pltpu.async_copy(pos_hbm.at[dv.at[b]], pd.at[b], gsems[b])

        def wait_gather(b):
            pltpu.make_async_copy(pos_hbm.at[sv.at[b]], ps.at[b],
                                  gsems[b]).wait()
            pltpu.make_async_copy(pos_hbm.at[dv.at[b]], pd.at[b],
                                  gsems[b]).wait()

        def halfstep(c, cur, oth, has_next, has_next2):
            @pl.when(has_next)
            def _():
                wait_stage(c + 1, oth)
                start_gather(oth)

            wait_gather(cur)

            def body(i, _):
                ps[cur, i] = pd[cur, i] - ps[cur, i]
                return 0

            lax.fori_loop(0, CH, body, 0, unroll=2)
            pltpu.sync_copy(ps.at[cur], out_hbm.at[pl.ds(cbase(c), CH)])

            @pl.when(has_next2)
            def _():
                stage(c + 2, cur)

        stage(0, 0)
        wait_stage(0, 0)
        start_gather(0)
        stage(1, 1)

        def pair(j, _):
            c0 = 2 * j
            halfstep(c0, 0, 1, c0 + 1 < KPW, c0 + 2 < KPW)
            halfstep(c0 + 1, 1, 0, c0 + 2 < KPW, c0 + 3 < KPW)
            return 0

        lax.fori_loop(0, KPW // 2, pair, 0)

    return k(pos_t, src, dst)


def _tc_edge(vec, shifts, m1_0, m2_0, m3_0, m1_1, m2_1, m3_1):
    def body(vec_ref, shf_ref, a1, a2, a3, b1, b2, b3, sh_ref, w0_ref, w1_ref):
        v = vec_ref[:, 0:3] + shf_ref[...]
        l2 = jnp.sum(v * v, axis=1) + 1e-12
        length = jnp.sqrt(l2)
        u = length * (1.0 / R_MAX)
        unit = v / length[:, None]
        s3 = 3.0 ** 0.5
        sh_ref[...] = jnp.concatenate(
            [jnp.ones((EBLK, 1), _f32), s3 * unit], axis=1)
        u2 = u * u
        u4 = u2 * u2
        u5 = u4 * u
        poly = 1.0 - 21.0 * u5 + 35.0 * u5 * u - 15.0 * u5 * u2
        cut = jnp.where(u < 1.0, poly, 0.0)
        nvec = (lax.broadcasted_iota(_i32, (1, NB), 1) + 1).astype(_f32)
        b = ((2.0 / R_MAX) ** 0.5) * jnp.sin(nvec * jnp.pi * u[:, None]) \
            / length[:, None]
        ef = b * cut[:, None]
        for (m1, m2, m3, out_ref) in ((a1, a2, a3, w0_ref),
                                      (b1, b2, b3, w1_ref)):
            x = jnp.dot(ef, m1[...], preferred_element_type=_f32)
            x = jax.nn.silu(x)
            x = jax.nn.silu(jnp.dot(x, m2[...], preferred_element_type=_f32))
            we = jnp.dot(x, m3[...], preferred_element_type=_f32) * MSG_SCALE
            for fb in range(FBN):
                out_ref[fb] = we[:, fb * 32:(fb + 1) * 32]

    wspec = [
        pl.BlockSpec((NB, 64), lambda i: (0, 0)),
        pl.BlockSpec((64, 64), lambda i: (0, 0)),
        pl.BlockSpec((64, F), lambda i: (0, 0)),
    ]
    return pl.pallas_call(
        body,
        grid=(EGRID,),
        in_specs=[
            pl.BlockSpec((EBLK, 16), lambda i: (i, 0)),
            pl.BlockSpec((EBLK, 3), lambda i: (i, 0)),
        ] + wspec + wspec,
        out_specs=[
            pl.BlockSpec((EBLK, 4), lambda i: (i, 0)),
            pl.BlockSpec((FBN, EBLK, 32), lambda i: (0, i, 0)),
            pl.BlockSpec((FBN, EBLK, 32), lambda i: (0, i, 0)),
        ],
        out_shape=[
            jax.ShapeDtypeStruct((E, 4), _f32),
            jax.ShapeDtypeStruct((FBN, E, 32), _f32),
            jax.ShapeDtypeStruct((FBN, E, 32), _f32),
        ],
    )(vec, shifts, m1_0, m2_0, m3_0, m1_1, m2_1, m3_1)


def _tc_embed(node_attrs, batch3, w_embed):
    def body(attr_ref, b3_ref, we_ref, h0_ref, seg_ref):
        attrs = attr_ref[...]
        h0 = jnp.dot(attrs, we_ref[...], preferred_element_type=_f32)
        for fb in range(FBN):
            h0_ref[fb] = h0[:, fb * 32:(fb + 1) * 32]
        bt = b3_ref[0, 0]
        oh = (bt[:, None] ==
              lax.broadcasted_iota(_i32, (NBLK, G), 1)).astype(_f32)
        sa = lax.dot_general(oh, attrs, (((0,), (0,)), ((), ())),
                             preferred_element_type=_f32)

        @pl.when(pl.program_id(0) == 0)
        def _():
            seg_ref[...] = jnp.zeros_like(seg_ref)

        seg_ref[...] += sa

    return pl.pallas_call(
        body,
        grid=(NGRID,),
        in_specs=[
            pl.BlockSpec((NBLK, Z), lambda i: (i, 0)),
            pl.BlockSpec((1, 1, NBLK), lambda i: (i, 0, 0)),
            pl.BlockSpec((Z, F), lambda i: (0, 0)),
        ],
        out_specs=[
            pl.BlockSpec((FBN, NBLK, 32), lambda i: (0, i, 0)),
            pl.BlockSpec((G, Z), lambda i: (0, 0)),
        ],
        out_shape=[
            jax.ShapeDtypeStruct((FBN, N, 32), _f32),
            jax.ShapeDtypeStruct((G, Z), _f32),
        ],
    )(node_attrs, batch3, w_embed)


def _sc_msg(h_tabs, sh, we, src, dst, first_layer):
    ht = 32 if first_layer else F
    mesh = plsc.VectorSubcoreMesh(core_axis_name="c", subcore_axis_name="s")

    @functools.partial(
        pl.kernel,
        out_type=jax.ShapeDtypeStruct((NC, FBN, N, F), _f32),
        mesh=mesh,
        scratch_types=[
            pltpu.VMEM_SHARED((N, F), _f32),
            pltpu.VMEM((2, MCH), _i32),
            pltpu.VMEM((4, MCH), _i32),
            pltpu.VMEM((2, MCH, ht), _f32),
            pltpu.VMEM((2, MCH, F), _f32),
            pltpu.VMEM((2, MCH, 32), _f32),
            pltpu.VMEM((2, MCH * 4 + 16), _f32),
            pltpu.SemaphoreType.DMA,
            pltpu.SemaphoreType.DMA,
            pltpu.SemaphoreType.DMA,
            pltpu.SemaphoreType.DMA,
            pltpu.SemaphoreType.DMA,
            pltpu.SemaphoreType.DMA,
        ],
        compiler_params=pltpu.CompilerParams(use_tc_tiling_on_sc=False),
    )
    def k(h0_hbm, h1_hbm, h2_hbm, h3_hbm, sh_hbm, we_hbm, src_hbm, dst_hbm,
          out_hbm, agg, sv, dv, rows, pay, wev, shv,
          ls0, ls1, gs0, gs1, ss0, ss1):
        cc = lax.axis_index("c")
        ss = lax.axis_index("s")
        wid = cc * NS + ss
        r0 = ss * NPT
        htabs = (h0_hbm, h1_hbm, h2_hbm, h3_hbm)
        lsems = (ls0, ls1)
        gsems = (gs0, gs1)
        ssems = (ss0, ss1)

        def cbase(c):
            return jnp.minimum(wid + c * NW, NMCH - 1) * MCH

        def is_real(c):
            return wid + c * NW < NMCH

        for fb in range(FBN):
            htab = htabs[fb]

            def stage(c, b):
                base = cbase(c)
                pltpu.async_copy(src_hbm.at[pl.ds(base, MCH)], sv.at[b],
                                 lsems[b])
                pltpu.async_copy(dst_hbm.at[pl.ds(base, MCH)], dv.at[c % 4],
                                 lsems[b])
                pltpu.async_copy(we_hbm.at[fb, pl.ds(base, MCH)], wev.at[b],
                                 lsems[b])
                pltpu.async_copy(sh_hbm.at[pl.ds(base * 4, MCH * 4)],
                                 shv.at[b, pl.ds(0, MCH * 4)], lsems[b])

            def wait_stage(c, b):
                base = cbase(c)
                pltpu.make_async_copy(src_hbm.at[pl.ds(base, MCH)], sv.at[b],
                                      lsems[b]).wait()
                pltpu.make_async_copy(dst_hbm.at[pl.ds(base, MCH)],
                                      dv.at[c % 4], lsems[b]).wait()
                pltpu.make_async_copy(we_hbm.at[fb, pl.ds(base, MCH)],
                                      wev.at[b], lsems[b]).wait()
                pltpu.make_async_copy(sh_hbm.at[pl.ds(base * 4, MCH * 4)],
                                      shv.at[b, pl.ds(0, MCH * 4)],
                                      lsems[b]).wait()

            def start_gather(b):
                pltpu.async_copy(htab.at[sv.at[b]], rows.at[b], gsems[b])

            def wait_gather(b):
                pltpu.make_async_copy(htab.at[sv.at[b]], rows.at[b],
                                      gsems[b]).wait()

            def start_scatter(c, b):
                @pl.when(is_real(c))
                def _():
                    pltpu.async_copy(pay.at[b], agg.at[dv.at[c % 4]],
                                     ssems[b], add=True)

            def wait_scatter(c, b):
                @pl.when(is_real(c))
                def _():
                    pltpu.make_async_copy(pay.at[b], agg.at[dv.at[c % 4]],
                                          ssems[b]).wait()

            def compute(b):
                def ebody(i, _):
                    svec = shv[b, pl.ds(i * 4, 16)]
                    s1 = svec[1]
                    s2 = svec[2]
                    s3 = svec[3]
                    for gg in range(2):
                        o = gg * 16
                        if first_layer:
                            proj = rows[b, i, pl.ds(o, 16)]
                        else:
                            proj = (rows[b, i, pl.ds(o, 16)]
                                    + rows[b, i, pl.ds(32 + o, 16)] * s1
                                    + rows[b, i, pl.ds(64 + o, 16)] * s2
                                    + rows[b, i, pl.ds(96 + o, 16)] * s3)
                        a = proj * wev[b, i, pl.ds(o, 16)]
                        pay[b, i, pl.ds(o, 16)] = a
                        pay[b, i, pl.ds(32 + o, 16)] = a * s1
                        pay[b, i, pl.ds(64 + o, 16)] = a * s2
                        pay[b, i, pl.ds(96 + o, 16)] = a * s3
                    return 0

                lax.fori_loop(0, MCH, ebody, 0, unroll=2)

            def zbody(r, _):
                for gg in range(F // 16):
                    pay[0, r, pl.ds(gg * 16, 16)] = jnp.zeros((16,), _f32)
                return 0

            lax.fori_loop(0, MCH, zbody, 0)
            off = 0
            for cnt in (64,) * 9 + (NPT - 576,):
                pltpu.sync_copy(pay.at[0, pl.ds(0, cnt)],
                                agg.at[pl.ds(r0 + off, cnt)])
                off += cnt
            plsc.subcore_barrier()

            def halfstep(c, cur, oth, has_next, has_next2):
                @pl.when(has_next)
                def _():
                    wait_stage(c + 1, oth)
                    start_gather(oth)

                wait_gather(cur)

                @pl.when(c >= 2)
                def _():
                    wait_scatter(c - 2, cur)

                compute(cur)
                start_scatter(c, cur)

                @pl.when(has_next2)
                def _():
                    stage(c + 2, cur)

            stage(0, 0)
            wait_stage(0, 0)
            start_gather(0)
            stage(1, 1)

            def pair(j, _):
                c0 = 2 * j
                halfstep(c0, 0, 1, c0 + 1 < KM, c0 + 2 < KM)
                halfstep(c0 + 1, 1, 0, c0 + 2 < KM, c0 + 3 < KM)
                return 0

            lax.fori_loop(0, KM // 2, pair, 0)
            wait_scatter(KM - 2, 0)
            wait_scatter(KM - 1, 1)
            plsc.subcore_barrier()

            off = 0
            for cnt in (128, 128, 128, 128, NPT - 512):
                for cc_s in range(NC):
                    @pl.when(cc == cc_s)
                    def _():
                        pltpu.sync_copy(
                            agg.at[pl.ds(r0 + off, cnt)],
                            out_hbm.at[cc_s, fb, pl.ds(r0 + off, cnt)])
                off += cnt
            plsc.subcore_barrier()

    return k(h_tabs[0], h_tabs[1], h_tabs[2], h_tabs[3], sh, we, src, dst)


def _tc_node(aggp, h_tab, batch3, wm, ws, first_layer):
    ht = 32 if first_layer else F

    def body(agg_ref, h_ref, b3_ref, wm_ref, ws_ref, hn_ref, seg_ref):
        a = agg_ref[0] + agg_ref[1]
        wmv = wm_ref[...]
        wsv = ws_ref[...]
        ms = []
        bsq = jnp.zeros((NBLK, F), _f32)
        for l in range(LM):
            x = jnp.concatenate(
                [a[fb, :, l * 32:(l + 1) * 32] for fb in range(FBN)], axis=1)
            m = jnp.dot(x, wmv, preferred_element_type=_f32)
            ms.append(m)
            bsq = bsq + m * m
        t = jnp.tanh(bsq)
        inv = None
        for l in range(LM):
            if first_layer:
                if l == 0:
                    hsrc = jnp.concatenate(
                        [h_ref[fb] for fb in range(FBN)], axis=1)
                    hx = jnp.dot(hsrc, wsv, preferred_element_type=_f32)
                else:
                    hx = None
            else:
                hsrc = jnp.concatenate(
                    [h_ref[fb, :, l * 32:(l + 1) * 32] for fb in range(FBN)],
                    axis=1)
                hx = jnp.dot(hsrc, wsv, preferred_element_type=_f32)
            hn = ms[l] + ms[l] * t
            if hx is not None:
                hn = hn + hx
            for fb in range(FBN):
                hn_ref[fb, :, l * 32:(l + 1) * 32] = hn[:, fb * 32:(fb + 1) * 32]
            if l == 0:
                inv = hn
        bt = b3_ref[0, 0]
        oh = (bt[:, None] ==
              lax.broadcasted_iota(_i32, (NBLK, G), 1)).astype(_f32)
        sa = lax.dot_general(oh, inv, (((0,), (0,)), ((), ())),
                             preferred_element_type=_f32)

        @pl.when(pl.program_id(0) == 0)
        def _():
            seg_ref[...] = jnp.zeros_like(seg_ref)

        seg_ref[...] += sa

    hspec = (pl.BlockSpec((FBN, NBLK, 32), lambda i: (0, i, 0)) if first_layer
             else pl.BlockSpec((FBN, NBLK, F), lambda i: (0, i, 0)))
    return pl.pallas_call(
        body,
        grid=(NGRID,),
        in_specs=[
            pl.BlockSpec((NC, FBN, NBLK, F), lambda i: (0, 0, i, 0)),
            hspec,
            pl.BlockSpec((1, 1, NBLK), lambda i: (i, 0, 0)),
            pl.BlockSpec((F, F), lambda i: (0, 0)),
            pl.BlockSpec((F, F), lambda i: (0, 0)),
        ],
        out_specs=[
            pl.BlockSpec((FBN, NBLK, F), lambda i: (0, i, 0)),
            pl.BlockSpec((G, F), lambda i: (0, 0)),
        ],
        out_shape=[
            jax.ShapeDtypeStruct((FBN, N, F), _f32),
            jax.ShapeDtypeStruct((G, F), _f32),
        ],
    )(aggp, h_tab, batch3, wm, ws)


def _tc_final(seg_attr, ae1, seg1, seg2, wr0, wi0, wr1, wi1, wdec):
    def body(sa_ref, ae_ref, s1_ref, s2_ref, wr0_ref, wi0_ref, wr1_ref,
             wi1_ref, wd_ref, out_ref):
        e0 = jnp.dot(sa_ref[...], ae_ref[...], preferred_element_type=_f32)
        s1 = s1_ref[...]
        s2 = s2_ref[...]
        en = (jnp.dot(s1, wr0_ref[...], preferred_element_type=_f32)
              + jnp.dot(s2, wr1_ref[...], preferred_element_type=_f32))
        il = (jnp.dot(s1, wi0_ref[...], preferred_element_type=_f32)
              + jnp.dot(s2, wi1_ref[...], preferred_element_type=_f32))
        out_ref[...] = e0 + en + jnp.dot(il, wd_ref[...],
                                         preferred_element_type=_f32)

    return pl.pallas_call(
        body,
        out_shape=jax.ShapeDtypeStruct((G, NE), _f32),
    )(seg_attr, ae1, seg1, seg2, wr0, wi0, wr1, wi1, wdec)


def kernel(positions, node_attrs, edge_index, shifts, batch, ptr,
           atomic_energies, W_embed, mlp1_0, mlp2_0, mlp3_0, Wmix_0, Wsc_0,
           Wread_0, Winv_0, mlp1_1, mlp2_1, mlp3_1, Wmix_1, Wsc_1, Wread_1,
           Winv_1, Wdec):
    src = edge_index[0].astype(_i32)
    dst = edge_index[1].astype(_i32)
    pos_t = jnp.zeros((N, 16), _f32).at[:, 0:3].set(positions.astype(_f32))
    batch3 = batch.astype(_i32).reshape(NGRID, 1, NBLK)

    vec = _sc_vec(pos_t, src, dst)
    sh, we0, we1 = _tc_edge(vec, shifts.astype(_f32), mlp1_0, mlp2_0, mlp3_0,
                            mlp1_1, mlp2_1, mlp3_1)
    h0_tab, seg_attr = _tc_embed(node_attrs.astype(_f32), batch3, W_embed)

    sh_flat = sh.reshape(E * 4)
    h0s = [h0_tab[fb] for fb in range(FBN)]
    aggp1 = _sc_msg(h0s, sh_flat, we0, src, dst, first_layer=True)
    h1_tab, seg1 = _tc_node(aggp1, h0_tab, batch3, Wmix_0, Wsc_0,
                            first_layer=True)

    h1s = [h1_tab[fb] for fb in range(FBN)]
    aggp2 = _sc_msg(h1s, sh_flat, we1, src, dst, first_layer=False)
    _, seg2 = _tc_node(aggp2, h1_tab, batch3, Wmix_1, Wsc_1,
                       first_layer=False)

    return _tc_final(seg_attr, atomic_energies.reshape(Z, 1).astype(_f32),
                     seg1, seg2, Wread_0, Winv_0, Wread_1, Winv_1, Wdec)


# speedup vs baseline: 2.9951x; 1.1106x over previous
import functools

import jax
import jax.numpy as jnp
from jax import lax
from jax.experimental import pallas as pl
from jax.experimental.pallas import tpu as pltpu
from jax.experimental.pallas import tpu_sc as plsc

N = 10000
E = 160000
F = 128
Z = 4
G = 16
NB = 8
LM = 4
NE = 3
NP = 4
R_MAX = 5.0
MSG_SCALE = 1.0 / (4.0 * 16.0)

NC = 2
NS = 16
NW = NC * NS
CH = 128
NCHT = E // CH
KPW = (NCHT + NW - 1) // NW
MCH = 64
NMCH = E // MCH
KM = 80
FBN = 4
NPT = N // NS

NBLK = 400
NGRID = N // NBLK
EBLK = 1280
EGRID = E // EBLK

_f32 = jnp.float32
_i32 = jnp.int32


def _sc_vec(pos_t, src, dst):
    mesh = plsc.VectorSubcoreMesh(core_axis_name="c", subcore_axis_name="s")

    @functools.partial(
        pl.kernel,
        out_type=jax.ShapeDtypeStruct((E, 16), _f32),
        mesh=mesh,
        scratch_types=[
            pltpu.VMEM((2, CH), _i32),
            pltpu.VMEM((2, CH), _i32),
            pltpu.VMEM((2, CH, 16), _f32),
            pltpu.VMEM((2, CH, 16), _f32),
            pltpu.SemaphoreType.DMA,
            pltpu.SemaphoreType.DMA,
            pltpu.SemaphoreType.DMA,
            pltpu.SemaphoreType.DMA,
        ],
        compiler_params=pltpu.CompilerParams(use_tc_tiling_on_sc=False),
    )
    def k(pos_hbm, src_hbm, dst_hbm, out_hbm, sv, dv, ps, pd,
          l0, l1, g0, g1):
        wid = lax.axis_index("c") * NS + lax.axis_index("s")
        lsems = (l0, l1)
        gsems = (g0, g1)

        def cbase(c):
            return jnp.minimum(wid + c * NW, NCHT - 1) * CH

        def stage(c, b):
            base = cbase(c)
            pltpu.async_copy(src_hbm.at[pl.ds(base, CH)], sv.at[b], lsems[b])
            pltpu.async_copy(dst_hbm.at[pl.ds(base, CH)], dv.at[b], lsems[b])

        def wait_stage(c, b):
            base = cbase(c)
            pltpu.make_async_copy(src_hbm.at[pl.ds(base, CH)], sv.at[b],
                                  lsems[b]).wait()
            pltpu.make_async_copy(dst_hbm.at[pl.ds(base, CH)], dv.at[b],
                                  lsems[b]).wait()

        def start_gather(b):
            pltpu.async_copy(pos_hbm.at[sv.at[b]], ps.at[b], gsems[b])
            pltpu.async_copy(pos_hbm.at[dv.at[b]], pd.at[b], gsems[b])

        def wait_gather(b):
            pltpu.make_async_copy(pos_hbm.at[sv.at[b]], ps.at[b],
                                  gsems[b]).wait()
            pltpu.make_async_copy(pos_hbm.at[dv.at[b]], pd.at[b],
                                  gsems[b]).wait()

        def halfstep(c, cur, oth, has_next, has_next2):
            @pl.when(has_next)
            def _():
                wait_stage(c + 1, oth)
                start_gather(oth)

            wait_gather(cur)

            def body(i, _):
                ps[cur, i] = pd[cur, i] - ps[cur, i]
                return 0

            lax.fori_loop(0, CH, body, 0, unroll=2)
            pltpu.sync_copy(ps.at[cur], out_hbm.at[pl.ds(cbase(c), CH)])

            @pl.when(has_next2)
            def _():
                stage(c + 2, cur)

        stage(0, 0)
        wait_stage(0, 0)
        start_gather(0)
        stage(1, 1)

        def pair(j, _):
            c0 = 2 * j
            halfstep(c0, 0, 1, c0 + 1 < KPW, c0 + 2 < KPW)
            halfstep(c0 + 1, 1, 0, c0 + 2 < KPW, c0 + 3 < KPW)
            return 0

        lax.fori_loop(0, KPW // 2, pair, 0)

    return k(pos_t, src, dst)


def _tc_edge(vec, shifts, m1_0, m2_0, m3_0, m1_1, m2_1, m3_1):
    def _eye(n):
        return (lax.broadcasted_iota(_i32, (n, n), 0) ==
                lax.broadcasted_iota(_i32, (n, n), 1)).astype(_f32)

    def _t(x, n):
        return lax.dot_general(_eye(n), x, (((1,), (1,)), ((), ())),
                               preferred_element_type=_f32)

    def body(vec_ref, shf_ref, a1, a2, a3, b1, b2, b3, sh_ref, w0_ref, w1_ref):
        vbt = _t(vec_ref[:, 0:3], 3) + _t(shf_ref[...], 3)
        vx = vbt[0:1]
        vy = vbt[1:2]
        vz = vbt[2:3]
        l2 = vx * vx + vy * vy + vz * vz + 1e-12
        length = jnp.sqrt(l2)
        rlen = 1.0 / length
        u = length * (1.0 / R_MAX)
        s3 = 3.0 ** 0.5
        sht = jnp.concatenate(
            [jnp.ones((1, EBLK), _f32), s3 * (vbt * rlen)], axis=0)
        sh_ref[...] = lax.dot_general(sht, _eye(4), (((0,), (0,)), ((), ())),
                                      preferred_element_type=_f32)
        u2 = u * u
        u4 = u2 * u2
        u5 = u4 * u
        poly = 1.0 - 21.0 * u5 + 35.0 * u5 * u - 15.0 * u5 * u2
        cut = jnp.where(u < 1.0, poly, 0.0)
        nvec = (lax.broadcasted_iota(_i32, (NB, 1), 0) + 1).astype(_f32)
        bt = ((2.0 / R_MAX) ** 0.5) * jnp.sin(nvec * (jnp.pi * u)) * rlen
        eft = bt * cut
        for (m1, m2, m3, out_ref) in ((a1, a2, a3, w0_ref),
                                      (b1, b2, b3, w1_ref)):
            x = lax.dot_general(m1[...], eft, (((0,), (0,)), ((), ())),
                                preferred_element_type=_f32)
            x = jax.nn.silu(x)
            x = jax.nn.silu(
                lax.dot_general(m2[...], x, (((0,), (0,)), ((), ())),
                                preferred_element_type=_f32))
            wet = lax.dot_general(m3[...], x, (((0,), (0,)), ((), ())),
                                  preferred_element_type=_f32) * MSG_SCALE
            we = lax.dot_general(wet, _eye(F), (((0,), (0,)), ((), ())),
                                 preferred_element_type=_f32)
            for fb in range(FBN):
                out_ref[fb] = we[:, fb * 32:(fb + 1) * 32]

    wspec = [
        pl.BlockSpec((NB, 64), lambda i: (0, 0)),
        pl.BlockSpec((64, 64), lambda i: (0, 0)),
        pl.BlockSpec((64, F), lambda i: (0, 0)),
    ]
    return pl.pallas_call(
        body,
        grid=(EGRID,),
        in_specs=[
            pl.BlockSpec((EBLK, 16), lambda i: (i, 0)),
            pl.BlockSpec((EBLK, 3), lambda i: (i, 0)),
        ] + wspec + wspec,
        out_specs=[
            pl.BlockSpec((EBLK, 4), lambda i: (i, 0)),
            pl.BlockSpec((FBN, EBLK, 32), lambda i: (0, i, 0)),
            pl.BlockSpec((FBN, EBLK, 32), lambda i: (0, i, 0)),
        ],
        out_shape=[
            jax.ShapeDtypeStruct((E, 4), _f32),
            jax.ShapeDtypeStruct((FBN, E, 32), _f32),
            jax.ShapeDtypeStruct((FBN, E, 32), _f32),
        ],
    )(vec, shifts, m1_0, m2_0, m3_0, m1_1, m2_1, m3_1)


def _tc_embed(node_attrs, batch3, w_embed):
    def body(attr_ref, b3_ref, we_ref, h0_ref, seg_ref):
        attrs = attr_ref[...]
        h0 = jnp.dot(attrs, we_ref[...], preferred_element_type=_f32)
        for fb in range(FBN):
            h0_ref[fb] = h0[:, fb * 32:(fb + 1) * 32]
        bt = b3_ref[0, 0]
        oh = (bt[:, None] ==
              lax.broadcasted_iota(_i32, (NBLK, G), 1)).astype(_f32)
        sa = lax.dot_general(oh, attrs, (((0,), (0,)), ((), ())),
                             preferred_element_type=_f32)

        @pl.when(pl.program_id(0) == 0)
        def _():
            seg_ref[...] = jnp.zeros_like(seg_ref)

        seg_ref[...] += sa

    return pl.pallas_call(
        body,
        grid=(NGRID,),
        in_specs=[
            pl.BlockSpec((NBLK, Z), lambda i: (i, 0)),
            pl.BlockSpec((1, 1, NBLK), lambda i: (i, 0, 0)),
            pl.BlockSpec((Z, F), lambda i: (0, 0)),
        ],
        out_specs=[
            pl.BlockSpec((FBN, NBLK, 32), lambda i: (0, i, 0)),
            pl.BlockSpec((G, Z), lambda i: (0, 0)),
        ],
        out_shape=[
            jax.ShapeDtypeStruct((FBN, N, 32), _f32),
            jax.ShapeDtypeStruct((G, Z), _f32),
        ],
    )(node_attrs, batch3, w_embed)


def _sc_msg(h_tabs, sh, we, src, dst, first_layer):
    ht = 32 if first_layer else F
    mesh = plsc.VectorSubcoreMesh(core_axis_name="c", subcore_axis_name="s")

    @functools.partial(
        pl.kernel,
        out_type=jax.ShapeDtypeStruct((NC, FBN, N, F), _f32),
        mesh=mesh,
        scratch_types=[
            pltpu.VMEM_SHARED((N, F), _f32),
            pltpu.VMEM((2, MCH), _i32),
            pltpu.VMEM((4, MCH), _i32),
            pltpu.VMEM((2, MCH, ht), _f32),
            pltpu.VMEM((2, MCH, F), _f32),
            pltpu.VMEM((2, MCH, 32), _f32),
            pltpu.VMEM((2, MCH * 4 + 16), _f32),
            pltpu.SemaphoreType.DMA,
            pltpu.SemaphoreType.DMA,
            pltpu.SemaphoreType.DMA,
            pltpu.SemaphoreType.DMA,
            pltpu.SemaphoreType.DMA,
            pltpu.SemaphoreType.DMA,
        ],
        compiler_params=pltpu.CompilerParams(use_tc_tiling_on_sc=False),
    )
    def k(h0_hbm, h1_hbm, h2_hbm, h3_hbm, sh_hbm, we_hbm, src_hbm, dst_hbm,
          out_hbm, agg, sv, dv, rows, pay, wev, shv,
          ls0, ls1, gs0, gs1, ss0, ss1):
        cc = lax.axis_index("c")
        ss = lax.axis_index("s")
        wid = cc * NS + ss
        r0 = ss * NPT
        htabs = (h0_hbm, h1_hbm, h2_hbm, h3_hbm)
        lsems = (ls0, ls1)
        gsems = (gs0, gs1)
        ssems = (ss0, ss1)

        def cbase(c):
            return jnp.minimum(wid + c * NW, NMCH - 1) * MCH

        def is_real(c):
            return wid + c * NW < NMCH

        for fb in range(FBN):
            htab = htabs[fb]

            def stage(c, b):
                base = cbase(c)
                pltpu.async_copy(src_hbm.at[pl.ds(base, MCH)], sv.at[b],
                                 lsems[b])
                pltpu.async_copy(dst_hbm.at[pl.ds(base, MCH)], dv.at[c % 4],
                                 lsems[b])
                pltpu.async_copy(we_hbm.at[fb, pl.ds(base, MCH)], wev.at[b],
                                 lsems[b])
                pltpu.async_copy(sh_hbm.at[pl.ds(base * 4, MCH * 4)],
                                 shv.at[b, pl.ds(0, MCH * 4)], lsems[b])

            def wait_stage(c, b):
                base = cbase(c)
                pltpu.make_async_copy(src_hbm.at[pl.ds(base, MCH)], sv.at[b],
                                      lsems[b]).wait()
                pltpu.make_async_copy(dst_hbm.at[pl.ds(base, MCH)],
                                      dv.at[c % 4], lsems[b]).wait()
                pltpu.make_async_copy(we_hbm.at[fb, pl.ds(base, MCH)],
                                      wev.at[b], lsems[b]).wait()
                pltpu.make_async_copy(sh_hbm.at[pl.ds(base * 4, MCH * 4)],
                                      shv.at[b, pl.ds(0, MCH * 4)],
                                      lsems[b]).wait()

            def start_gather(b):
                pltpu.async_copy(htab.at[sv.at[b]], rows.at[b], gsems[b])

            def wait_gather(b):
                pltpu.make_async_copy(htab.at[sv.at[b]], rows.at[b],
                                      gsems[b]).wait()

            def start_scatter(c, b):
                @pl.when(is_real(c))
                def _():
                    pltpu.async_copy(pay.at[b], agg.at[dv.at[c % 4]],
                                     ssems[b], add=True)

            def wait_scatter(c, b):
                @pl.when(is_real(c))
                def _():
                    pltpu.make_async_copy(pay.at[b], agg.at[dv.at[c % 4]],
                                          ssems[b]).wait()

            def compute(b):
                def ebody(i, _):
                    svec = shv[b, pl.ds(i * 4, 16)]
                    s1 = svec[1]
                    s2 = svec[2]
                    s3 = svec[3]
                    for gg in range(2):
                        o = gg * 16
                        if first_layer:
                            proj = rows[b, i, pl.ds(o, 16)]
                        else:
                            proj = (rows[b, i, pl.ds(o, 16)]
                                    + rows[b, i, pl.ds(32 + o, 16)] * s1
                                    + rows[b, i, pl.ds(64 + o, 16)] * s2
                                    + rows[b, i, pl.ds(96 + o, 16)] * s3)
                        a = proj * wev[b, i, pl.ds(o, 16)]
                        pay[b, i, pl.ds(o, 16)] = a
                        pay[b, i, pl.ds(32 + o, 16)] = a * s1
                        pay[b, i, pl.ds(64 + o, 16)] = a * s2
                        pay[b, i, pl.ds(96 + o, 16)] = a * s3
                    return 0

                lax.fori_loop(0, MCH, ebody, 0, unroll=2)

            def zbody(r, _):
                for gg in range(F // 16):
                    pay[0, r, pl.ds(gg * 16, 16)] = jnp.zeros((16,), _f32)
                return 0

            lax.fori_loop(0, MCH, zbody, 0)
            off = 0
            for cnt in (64,) * 9 + (NPT - 576,):
                pltpu.sync_copy(pay.at[0, pl.ds(0, cnt)],
                                agg.at[pl.ds(r0 + off, cnt)])
                off += cnt
            plsc.subcore_barrier()

            def halfstep(c, cur, oth, has_next, has_next2):
                @pl.when(has_next)
                def _():
                    wait_stage(c + 1, oth)
                    start_gather(oth)

                wait_gather(cur)

                @pl.when(c >= 2)
                def _():
                    wait_scatter(c - 2, cur)

                compute(cur)
                start_scatter(c, cur)

                @pl.when(has_next2)
                def _():
                    stage(c + 2, cur)

            stage(0, 0)
            wait_stage(0, 0)
            start_gather(0)
            stage(1, 1)

            def pair(j, _):
                c0 = 2 * j
                halfstep(c0, 0, 1, c0 + 1 < KM, c0 + 2 < KM)
                halfstep(c0 + 1, 1, 0, c0 + 2 < KM, c0 + 3 < KM)
                return 0

            lax.fori_loop(0, KM // 2, pair, 0)
            wait_scatter(KM - 2, 0)
            wait_scatter(KM - 1, 1)
            plsc.subcore_barrier()

            off = 0
            for cnt in (128, 128, 128, 128, NPT - 512):
                for cc_s in range(NC):
                    @pl.when(cc == cc_s)
                    def _():
                        pltpu.sync_copy(
                            agg.at[pl.ds(r0 + off, cnt)],
                            out_hbm.at[cc_s, fb, pl.ds(r0 + off, cnt)])
                off += cnt
            plsc.subcore_barrier()

    return k(h_tabs[0], h_tabs[1], h_tabs[2], h_tabs[3], sh, we, src, dst)


def _tc_node(aggp, h_tab, batch3, wm, ws, first_layer):
    ht = 32 if first_layer else F

    def body(agg_ref, h_ref, b3_ref, wm_ref, ws_ref, hn_ref, seg_ref):
        a = agg_ref[0] + agg_ref[1]
        wmv = wm_ref[...]
        wsv = ws_ref[...]
        ms = []
        bsq = jnp.zeros((NBLK, F), _f32)
        for l in range(LM):
            x = jnp.concatenate(
                [a[fb, :, l * 32:(l + 1) * 32] for fb in range(FBN)], axis=1)
            m = jnp.dot(x, wmv, preferred_element_type=_f32)
            ms.append(m)
            bsq = bsq + m * m
        t = jnp.tanh(bsq)
        inv = None
        for l in range(LM):
            if first_layer:
                if l == 0:
                    hsrc = jnp.concatenate(
                        [h_ref[fb] for fb in range(FBN)], axis=1)
                    hx = jnp.dot(hsrc, wsv, preferred_element_type=_f32)
                else:
                    hx = None
            else:
                hsrc = jnp.concatenate(
                    [h_ref[fb, :, l * 32:(l + 1) * 32] for fb in range(FBN)],
                    axis=1)
                hx = jnp.dot(hsrc, wsv, preferred_element_type=_f32)
            hn = ms[l] + ms[l] * t
            if hx is not None:
                hn = hn + hx
            for fb in range(FBN):
                hn_ref[fb, :, l * 32:(l + 1) * 32] = hn[:, fb * 32:(fb + 1) * 32]
            if l == 0:
                inv = hn
        bt = b3_ref[0, 0]
        oh = (bt[:, None] ==
              lax.broadcasted_iota(_i32, (NBLK, G), 1)).astype(_f32)
        sa = lax.dot_general(oh, inv, (((0,), (0,)), ((), ())),
                             preferred_element_type=_f32)

        @pl.when(pl.program_id(0) == 0)
        def _():
            seg_ref[...] = jnp.zeros_like(seg_ref)

        seg_ref[...] += sa

    hspec = (pl.BlockSpec((FBN, NBLK, 32), lambda i: (0, i, 0)) if first_layer
             else pl.BlockSpec((FBN, NBLK, F), lambda i: (0, i, 0)))
    return pl.pallas_call(
        body,
        grid=(NGRID,),
        in_specs=[
            pl.BlockSpec((NC, FBN, NBLK, F), lambda i: (0, 0, i, 0)),
            hspec,
            pl.BlockSpec((1, 1, NBLK), lambda i: (i, 0, 0)),
            pl.BlockSpec((F, F), lambda i: (0, 0)),
            pl.BlockSpec((F, F), lambda i: (0, 0)),
        ],
        out_specs=[
            pl.BlockSpec((FBN, NBLK, F), lambda i: (0, i, 0)),
            pl.BlockSpec((G, F), lambda i: (0, 0)),
        ],
        out_shape=[
            jax.ShapeDtypeStruct((FBN, N, F), _f32),
            jax.ShapeDtypeStruct((G, F), _f32),
        ],
    )(aggp, h_tab, batch3, wm, ws)


def _tc_final(seg_attr, ae1, seg1, seg2, wr0, wi0, wr1, wi1, wdec):
    def body(sa_ref, ae_ref, s1_ref, s2_ref, wr0_ref, wi0_ref, wr1_ref,
             wi1_ref, wd_ref, out_ref):
        e0 = jnp.dot(sa_ref[...], ae_ref[...], preferred_element_type=_f32)
        s1 = s1_ref[...]
        s2 = s2_ref[...]
        en = (jnp.dot(s1, wr0_ref[...], preferred_element_type=_f32)
              + jnp.dot(s2, wr1_ref[...], preferred_element_type=_f32))
        il = (jnp.dot(s1, wi0_ref[...], preferred_element_type=_f32)
              + jnp.dot(s2, wi1_ref[...], preferred_element_type=_f32))
        out_ref[...] = e0 + en + jnp.dot(il, wd_ref[...],
                                         preferred_element_type=_f32)

    return pl.pallas_call(
        body,
        out_shape=jax.ShapeDtypeStruct((G, NE), _f32),
    )(seg_attr, ae1, seg1, seg2, wr0, wi0, wr1, wi1, wdec)


def kernel(positions, node_attrs, edge_index, shifts, batch, ptr,
           atomic_energies, W_embed, mlp1_0, mlp2_0, mlp3_0, Wmix_0, Wsc_0,
           Wread_0, Winv_0, mlp1_1, mlp2_1, mlp3_1, Wmix_1, Wsc_1, Wread_1,
           Winv_1, Wdec):
    src = edge_index[0].astype(_i32)
    dst = edge_index[1].astype(_i32)
    pos_t = jnp.zeros((N, 16), _f32).at[:, 0:3].set(positions.astype(_f32))
    batch3 = batch.astype(_i32).reshape(NGRID, 1, NBLK)

    vec = _sc_vec(pos_t, src, dst)
    sh, we0, we1 = _tc_edge(vec, shifts.astype(_f32), mlp1_0, mlp2_0, mlp3_0,
                            mlp1_1, mlp2_1, mlp3_1)
    h0_tab, seg_attr = _tc_embed(node_attrs.astype(_f32), batch3, W_embed)

    sh_flat = sh.reshape(E * 4)
    h0s = [h0_tab[fb] for fb in range(FBN)]
    aggp1 = _sc_msg(h0s, sh_flat, we0, src, dst, first_layer=True)
    h1_tab, seg1 = _tc_node(aggp1, h0_tab, batch3, Wmix_0, Wsc_0,
                            first_layer=True)

    h1s = [h1_tab[fb] for fb in range(FBN)]
    aggp2 = _sc_msg(h1s, sh_flat, we1, src, dst, first_layer=False)
    _, seg2 = _tc_node(aggp2, h1_tab, batch3, Wmix_1, Wsc_1,
                       first_layer=False)

    return _tc_final(seg_attr, atomic_energies.reshape(Z, 1).astype(_f32),
                     seg1, seg2, Wread_0, Winv_0, Wread_1, Winv_1, Wdec)


# speedup vs baseline: 3.0313x; 1.0121x over previous
import functools

import jax
import jax.numpy as jnp
from jax import lax
from jax.experimental import pallas as pl
from jax.experimental.pallas import tpu as pltpu
from jax.experimental.pallas import tpu_sc as plsc

N = 10000
E = 160000
F = 128
Z = 4
G = 16
NB = 8
LM = 4
NE = 3
NP = 4
R_MAX = 5.0
MSG_SCALE = 1.0 / (4.0 * 16.0)

NC = 2
NS = 16
NW = NC * NS
CH = 128
NCHT = E // CH
KPW = (NCHT + NW - 1) // NW
MCH = 64
NMCH = E // MCH
KM = 80
FBN = 4
NPT = N // NS

NBLK = 400
NGRID = N // NBLK
EBLK = 1280
EGRID = E // EBLK

_f32 = jnp.float32
_i32 = jnp.int32


def _sc_vec(pos_t, src, dst):
    mesh = plsc.VectorSubcoreMesh(core_axis_name="c", subcore_axis_name="s")

    @functools.partial(
        pl.kernel,
        out_type=jax.ShapeDtypeStruct((E, 16), _f32),
        mesh=mesh,
        scratch_types=[
            pltpu.VMEM((2, CH), _i32),
            pltpu.VMEM((2, CH), _i32),
            pltpu.VMEM((2, CH, 16), _f32),
            pltpu.VMEM((2, CH, 16), _f32),
            pltpu.SemaphoreType.DMA,
            pltpu.SemaphoreType.DMA,
            pltpu.SemaphoreType.DMA,
            pltpu.SemaphoreType.DMA,
        ],
        compiler_params=pltpu.CompilerParams(use_tc_tiling_on_sc=False),
    )
    def k(pos_hbm, src_hbm, dst_hbm, out_hbm, sv, dv, ps, pd,
          l0, l1, g0, g1):
        wid = lax.axis_index("c") * NS + lax.axis_index("s")
        lsems = (l0, l1)
        gsems = (g0, g1)

        def cbase(c):
            return jnp.minimum(wid + c * NW, NCHT - 1) * CH

        def stage(c, b):
            base = cbase(c)
            pltpu.async_copy(src_hbm.at[pl.ds(base, CH)], sv.at[b], lsems[b])
            pltpu.async_copy(dst_hbm.at[pl.ds(base, CH)], dv.at[b], lsems[b])

        def wait_stage(c, b):
            base = cbase(c)
            pltpu.make_async_copy(src_hbm.at[pl.ds(base, CH)], sv.at[b],
                                  lsems[b]).wait()
            pltpu.make_async_copy(dst_hbm.at[pl.ds(base, CH)], dv.at[b],
                                  lsems[b]).wait()

        def start_gather(b):
            pltpu.async_copy(pos_hbm.at[sv.at[b]], ps.at[b], gsems[b])
            pltpu.async_copy(pos_hbm.at[dv.at[b]], pd.at[b], gsems[b])

        def wait_gather(b):
            pltpu.make_async_copy(pos_hbm.at[sv.at[b]], ps.at[b],
                                  gsems[b]).wait()
            pltpu.make_async_copy(pos_hbm.at[dv.at[b]], pd.at[b],
                                  gsems[b]).wait()

        def halfstep(c, cur, oth, has_next, has_next2):
            @pl.when(has_next)
            def _():
                wait_stage(c + 1, oth)
                start_gather(oth)

            wait_gather(cur)

            def body(i, _):
                ps[cur, i] = pd[cur, i] - ps[cur, i]
                return 0

            lax.fori_loop(0, CH, body, 0, unroll=2)
            pltpu.sync_copy(ps.at[cur], out_hbm.at[pl.ds(cbase(c), CH)])

            @pl.when(has_next2)
            def _():
                stage(c + 2, cur)

        stage(0, 0)
        wait_stage(0, 0)
        start_gather(0)
        stage(1, 1)

        def pair(j, _):
            c0 = 2 * j
            halfstep(c0, 0, 1, c0 + 1 < KPW, c0 + 2 < KPW)
            halfstep(c0 + 1, 1, 0, c0 + 2 < KPW, c0 + 3 < KPW)
            return 0

        lax.fori_loop(0, KPW // 2, pair, 0)

    return k(pos_t, src, dst)


def _tc_edge(vec, shifts, m1_0, m2_0, m3_0, m1_1, m2_1, m3_1):
    def _eye(n):
        return (lax.broadcasted_iota(_i32, (n, n), 0) ==
                lax.broadcasted_iota(_i32, (n, n), 1)).astype(_f32)

    def _t(x, n):
        return lax.dot_general(_eye(n), x, (((1,), (1,)), ((), ())),
                               preferred_element_type=_f32)

    def body(vec_ref, shf_ref, a1, a2, a3, b1, b2, b3, sh_ref, w0_ref, w1_ref):
        vbt = _t(vec_ref[:, 0:3], 3) + _t(shf_ref[...], 3)
        vx = vbt[0:1]
        vy = vbt[1:2]
        vz = vbt[2:3]
        l2 = vx * vx + vy * vy + vz * vz + 1e-12
        length = jnp.sqrt(l2)
        rlen = 1.0 / length
        u = length * (1.0 / R_MAX)
        s3 = 3.0 ** 0.5
        sht = jnp.concatenate(
            [jnp.ones((1, EBLK), _f32), s3 * (vbt * rlen)], axis=0)
        sh_ref[...] = lax.dot_general(sht, _eye(4), (((0,), (0,)), ((), ())),
                                      preferred_element_type=_f32)
        u2 = u * u
        u4 = u2 * u2
        u5 = u4 * u
        poly = 1.0 - 21.0 * u5 + 35.0 * u5 * u - 15.0 * u5 * u2
        cut = jnp.where(u < 1.0, poly, 0.0)
        nvec = (lax.broadcasted_iota(_i32, (NB, 1), 0) + 1).astype(_f32)
        bt = ((2.0 / R_MAX) ** 0.5) * jnp.sin(nvec * (jnp.pi * u)) * rlen
        eft = bt * cut
        for (m1, m2, m3, out_ref) in ((a1, a2, a3, w0_ref),
                                      (b1, b2, b3, w1_ref)):
            x = lax.dot_general(m1[...], eft, (((0,), (0,)), ((), ())),
                                preferred_element_type=_f32)
            x = jax.nn.silu(x)
            x = jax.nn.silu(
                lax.dot_general(m2[...], x, (((0,), (0,)), ((), ())),
                                preferred_element_type=_f32))
            we = lax.dot_general(x, m3[...], (((0,), (0,)), ((), ())),
                                 preferred_element_type=_f32) * MSG_SCALE
            for fb in range(FBN):
                out_ref[fb] = we[:, fb * 32:(fb + 1) * 32]

    wspec = [
        pl.BlockSpec((NB, 64), lambda i: (0, 0)),
        pl.BlockSpec((64, 64), lambda i: (0, 0)),
        pl.BlockSpec((64, F), lambda i: (0, 0)),
    ]
    return pl.pallas_call(
        body,
        grid=(EGRID,),
        in_specs=[
            pl.BlockSpec((EBLK, 16), lambda i: (i, 0)),
            pl.BlockSpec((EBLK, 3), lambda i: (i, 0)),
        ] + wspec + wspec,
        out_specs=[
            pl.BlockSpec((EBLK, 4), lambda i: (i, 0)),
            pl.BlockSpec((FBN, EBLK, 32), lambda i: (0, i, 0)),
            pl.BlockSpec((FBN, EBLK, 32), lambda i: (0, i, 0)),
        ],
        out_shape=[
            jax.ShapeDtypeStruct((E, 4), _f32),
            jax.ShapeDtypeStruct((FBN, E, 32), _f32),
            jax.ShapeDtypeStruct((FBN, E, 32), _f32),
        ],
    )(vec, shifts, m1_0, m2_0, m3_0, m1_1, m2_1, m3_1)


def _tc_embed(node_attrs, batch3, w_embed):
    def body(attr_ref, b3_ref, we_ref, h0_ref, seg_ref):
        attrs = attr_ref[...]
        h0 = jnp.dot(attrs, we_ref[...], preferred_element_type=_f32)
        for fb in range(FBN):
            h0_ref[fb] = h0[:, fb * 32:(fb + 1) * 32]
        bt = b3_ref[0, 0]
        oh = (bt[:, None] ==
              lax.broadcasted_iota(_i32, (NBLK, G), 1)).astype(_f32)
        sa = lax.dot_general(oh, attrs, (((0,), (0,)), ((), ())),
                             preferred_element_type=_f32)

        @pl.when(pl.program_id(0) == 0)
        def _():
            seg_ref[...] = jnp.zeros_like(seg_ref)

        seg_ref[...] += sa

    return pl.pallas_call(
        body,
        grid=(NGRID,),
        in_specs=[
            pl.BlockSpec((NBLK, Z), lambda i: (i, 0)),
            pl.BlockSpec((1, 1, NBLK), lambda i: (i, 0, 0)),
            pl.BlockSpec((Z, F), lambda i: (0, 0)),
        ],
        out_specs=[
            pl.BlockSpec((FBN, NBLK, 32), lambda i: (0, i, 0)),
            pl.BlockSpec((G, Z), lambda i: (0, 0)),
        ],
        out_shape=[
            jax.ShapeDtypeStruct((FBN, N, 32), _f32),
            jax.ShapeDtypeStruct((G, Z), _f32),
        ],
    )(node_attrs, batch3, w_embed)


def _sc_msg(h_flat, sh, we, src, dst, first_layer):
    ht = 32 if first_layer else F
    mesh = plsc.VectorSubcoreMesh(core_axis_name="c", subcore_axis_name="s")

    @functools.partial(
        pl.kernel,
        out_type=jax.ShapeDtypeStruct((NC, FBN, N, F), _f32),
        mesh=mesh,
        scratch_types=[
            pltpu.VMEM_SHARED((N, F), _f32),
            pltpu.VMEM((2, MCH), _i32),
            pltpu.VMEM((4, MCH), _i32),
            pltpu.VMEM((2, MCH, ht), _f32),
            pltpu.VMEM((2, MCH, F), _f32),
            pltpu.VMEM((2, MCH, 32), _f32),
            pltpu.VMEM((2, MCH * 4 + 16), _f32),
            pltpu.SemaphoreType.DMA,
            pltpu.SemaphoreType.DMA,
            pltpu.SemaphoreType.DMA,
            pltpu.SemaphoreType.DMA,
            pltpu.SemaphoreType.DMA,
            pltpu.SemaphoreType.DMA,
        ],
        compiler_params=pltpu.CompilerParams(use_tc_tiling_on_sc=False),
    )
    def k(h_hbm, sh_hbm, we_hbm, src_hbm, dst_hbm,
          out_hbm, agg, sv, dv, rows, pay, wev, shv,
          ls0, ls1, gs0, gs1, ss0, ss1):
        cc = lax.axis_index("c")
        ss = lax.axis_index("s")
        wid = cc * NS + ss
        r0 = ss * NPT
        lsems = (ls0, ls1)
        gsems = (gs0, gs1)
        ssems = (ss0, ss1)

        def cbase(c):
            return jnp.minimum(wid + c * NW, NMCH - 1) * MCH

        def is_real(c):
            return wid + c * NW < NMCH

        for fb in range(FBN):

            def stage(c, b):
                base = cbase(c)
                pltpu.async_copy(src_hbm.at[pl.ds(base, MCH)], sv.at[b],
                                 lsems[b])
                pltpu.async_copy(dst_hbm.at[pl.ds(base, MCH)], dv.at[c % 4],
                                 lsems[b])
                pltpu.async_copy(we_hbm.at[fb, pl.ds(base, MCH)], wev.at[b],
                                 lsems[b])
                pltpu.async_copy(sh_hbm.at[pl.ds(base * 4, MCH * 4)],
                                 shv.at[b, pl.ds(0, MCH * 4)], lsems[b])

            def wait_stage(c, b):
                base = cbase(c)
                pltpu.make_async_copy(src_hbm.at[pl.ds(base, MCH)], sv.at[b],
                                      lsems[b]).wait()
                pltpu.make_async_copy(dst_hbm.at[pl.ds(base, MCH)],
                                      dv.at[c % 4], lsems[b]).wait()
                pltpu.make_async_copy(we_hbm.at[fb, pl.ds(base, MCH)],
                                      wev.at[b], lsems[b]).wait()
                pltpu.make_async_copy(sh_hbm.at[pl.ds(base * 4, MCH * 4)],
                                      shv.at[b, pl.ds(0, MCH * 4)],
                                      lsems[b]).wait()

            def adjust_idx(b):
                if fb > 0:
                    for g4 in range(MCH // 16):
                        sv[b, pl.ds(g4 * 16, 16)] = (
                            sv[b, pl.ds(g4 * 16, 16)] + fb * N)

            def start_gather(b):
                pltpu.async_copy(h_hbm.at[sv.at[b]], rows.at[b], gsems[b])

            def wait_gather(b):
                pltpu.make_async_copy(h_hbm.at[sv.at[b]], rows.at[b],
                                      gsems[b]).wait()

            def start_scatter(c, b):
                @pl.when(is_real(c))
                def _():
                    pltpu.async_copy(pay.at[b], agg.at[dv.at[c % 4]],
                                     ssems[b], add=True)

            def wait_scatter(c, b):
                @pl.when(is_real(c))
                def _():
                    pltpu.make_async_copy(pay.at[b], agg.at[dv.at[c % 4]],
                                          ssems[b]).wait()

            def compute(b):
                def ebody(i, _):
                    svec = shv[b, pl.ds(i * 4, 16)]
                    s1 = svec[1]
                    s2 = svec[2]
                    s3 = svec[3]
                    for gg in range(2):
                        o = gg * 16
                        if first_layer:
                            proj = rows[b, i, pl.ds(o, 16)]
                        else:
                            proj = (rows[b, i, pl.ds(o, 16)]
                                    + rows[b, i, pl.ds(32 + o, 16)] * s1
                                    + rows[b, i, pl.ds(64 + o, 16)] * s2
                                    + rows[b, i, pl.ds(96 + o, 16)] * s3)
                        a = proj * wev[b, i, pl.ds(o, 16)]
                        pay[b, i, pl.ds(o, 16)] = a
                        pay[b, i, pl.ds(32 + o, 16)] = a * s1
                        pay[b, i, pl.ds(64 + o, 16)] = a * s2
                        pay[b, i, pl.ds(96 + o, 16)] = a * s3
                    return 0

                lax.fori_loop(0, MCH, ebody, 0, unroll=2)

            def zbody(r, _):
                for gg in range(F // 16):
                    pay[0, r, pl.ds(gg * 16, 16)] = jnp.zeros((16,), _f32)
                return 0

            lax.fori_loop(0, MCH, zbody, 0)
            off = 0
            for cnt in (64,) * 9 + (NPT - 576,):
                pltpu.sync_copy(pay.at[0, pl.ds(0, cnt)],
                                agg.at[pl.ds(r0 + off, cnt)])
                off += cnt
            plsc.subcore_barrier()

            def halfstep(c, cur, oth, has_next, has_next2):
                @pl.when(has_next)
                def _():
                    wait_stage(c + 1, oth)
                    adjust_idx(oth)
                    start_gather(oth)

                wait_gather(cur)

                @pl.when(c >= 2)
                def _():
                    wait_scatter(c - 2, cur)

                compute(cur)
                start_scatter(c, cur)

                @pl.when(has_next2)
                def _():
                    stage(c + 2, cur)

            stage(0, 0)
            wait_stage(0, 0)
            adjust_idx(0)
            start_gather(0)
            stage(1, 1)

            def pair(j, _):
                c0 = 2 * j
                halfstep(c0, 0, 1, c0 + 1 < KM, c0 + 2 < KM)
                halfstep(c0 + 1, 1, 0, c0 + 2 < KM, c0 + 3 < KM)
                return 0

            lax.fori_loop(0, KM // 2, pair, 0)
            wait_scatter(KM - 2, 0)
            wait_scatter(KM - 1, 1)
            plsc.subcore_barrier()

            off = 0
            for cnt in (128, 128, 128, 128, NPT - 512):
                for cc_s in range(NC):
                    @pl.when(cc == cc_s)
                    def _():
                        pltpu.sync_copy(
                            agg.at[pl.ds(r0 + off, cnt)],
                            out_hbm.at[cc_s, fb, pl.ds(r0 + off, cnt)])
                off += cnt
            plsc.subcore_barrier()

    return k(h_flat, sh, we, src, dst)


def _tc_node(aggp, h_tab, batch3, wm, ws, first_layer):
    ht = 32 if first_layer else F

    def body(agg_ref, h_ref, b3_ref, wm_ref, ws_ref, hn_ref, seg_ref):
        a = agg_ref[0] + agg_ref[1]
        wmv = wm_ref[...]
        wsv = ws_ref[...]
        ms = []
        bsq = jnp.zeros((NBLK, F), _f32)
        for l in range(LM):
            x = jnp.concatenate(
                [a[fb, :, l * 32:(l + 1) * 32] for fb in range(FBN)], axis=1)
            m = jnp.dot(x, wmv, preferred_element_type=_f32)
            ms.append(m)
            bsq = bsq + m * m
        t = jnp.tanh(bsq)
        inv = None
        for l in range(LM):
            if first_layer:
                if l == 0:
                    hsrc = jnp.concatenate(
                        [h_ref[fb] for fb in range(FBN)], axis=1)
                    hx = jnp.dot(hsrc, wsv, preferred_element_type=_f32)
                else:
                    hx = None
            else:
                hsrc = jnp.concatenate(
                    [h_ref[fb, :, l * 32:(l + 1) * 32] for fb in range(FBN)],
                    axis=1)
                hx = jnp.dot(hsrc, wsv, preferred_element_type=_f32)
            hn = ms[l] + ms[l] * t
            if hx is not None:
                hn = hn + hx
            for fb in range(FBN):
                hn_ref[fb, :, l * 32:(l + 1) * 32] = hn[:, fb * 32:(fb + 1) * 32]
            if l == 0:
                inv = hn
        bt = b3_ref[0, 0]
        oh = (bt[:, None] ==
              lax.broadcasted_iota(_i32, (NBLK, G), 1)).astype(_f32)
        sa = lax.dot_general(oh, inv, (((0,), (0,)), ((), ())),
                             preferred_element_type=_f32)

        @pl.when(pl.program_id(0) == 0)
        def _():
            seg_ref[...] = jnp.zeros_like(seg_ref)

        seg_ref[...] += sa

    hspec = (pl.BlockSpec((FBN, NBLK, 32), lambda i: (0, i, 0)) if first_layer
             else pl.BlockSpec((FBN, NBLK, F), lambda i: (0, i, 0)))
    return pl.pallas_call(
        body,
        grid=(NGRID,),
        in_specs=[
            pl.BlockSpec((NC, FBN, NBLK, F), lambda i: (0, 0, i, 0)),
            hspec,
            pl.BlockSpec((1, 1, NBLK), lambda i: (i, 0, 0)),
            pl.BlockSpec((F, F), lambda i: (0, 0)),
            pl.BlockSpec((F, F), lambda i: (0, 0)),
        ],
        out_specs=[
            pl.BlockSpec((FBN, NBLK, F), lambda i: (0, i, 0)),
            pl.BlockSpec((G, F), lambda i: (0, 0)),
        ],
        out_shape=[
            jax.ShapeDtypeStruct((FBN, N, F), _f32),
            jax.ShapeDtypeStruct((G, F), _f32),
        ],
    )(aggp, h_tab, batch3, wm, ws)


def _tc_final(seg_attr, ae1, seg1, seg2, wr0, wi0, wr1, wi1, wdec):
    def body(sa_ref, ae_ref, s1_ref, s2_ref, wr0_ref, wi0_ref, wr1_ref,
             wi1_ref, wd_ref, out_ref):
        e0 = jnp.dot(sa_ref[...], ae_ref[...], preferred_element_type=_f32)
        s1 = s1_ref[...]
        s2 = s2_ref[...]
        en = (jnp.dot(s1, wr0_ref[...], preferred_element_type=_f32)
              + jnp.dot(s2, wr1_ref[...], preferred_element_type=_f32))
        il = (jnp.dot(s1, wi0_ref[...], preferred_element_type=_f32)
              + jnp.dot(s2, wi1_ref[...], preferred_element_type=_f32))
        out_ref[...] = e0 + en + jnp.dot(il, wd_ref[...],
                                         preferred_element_type=_f32)

    return pl.pallas_call(
        body,
        out_shape=jax.ShapeDtypeStruct((G, NE), _f32),
    )(seg_attr, ae1, seg1, seg2, wr0, wi0, wr1, wi1, wdec)


def kernel(positions, node_attrs, edge_index, shifts, batch, ptr,
           atomic_energies, W_embed, mlp1_0, mlp2_0, mlp3_0, Wmix_0, Wsc_0,
           Wread_0, Winv_0, mlp1_1, mlp2_1, mlp3_1, Wmix_1, Wsc_1, Wread_1,
           Winv_1, Wdec):
    src = edge_index[0].astype(_i32)
    dst = edge_index[1].astype(_i32)
    pos_t = jnp.zeros((N, 16), _f32).at[:, 0:3].set(positions.astype(_f32))
    batch3 = batch.astype(_i32).reshape(NGRID, 1, NBLK)

    vec = _sc_vec(pos_t, src, dst)
    sh, we0, we1 = _tc_edge(vec, shifts.astype(_f32), mlp1_0, mlp2_0, mlp3_0,
                            mlp1_1, mlp2_1, mlp3_1)
    h0_tab, seg_attr = _tc_embed(node_attrs.astype(_f32), batch3, W_embed)

    sh_flat = sh.reshape(E * 4)
    aggp1 = _sc_msg(h0_tab.reshape(FBN * N, 32), sh_flat, we0, src, dst,
                    first_layer=True)
    h1_tab, seg1 = _tc_node(aggp1, h0_tab, batch3, Wmix_0, Wsc_0,
                            first_layer=True)

    aggp2 = _sc_msg(h1_tab.reshape(FBN * N, F), sh_flat, we1, src, dst,
                    first_layer=False)
    _, seg2 = _tc_node(aggp2, h1_tab, batch3, Wmix_1, Wsc_1,
                       first_layer=False)

    return _tc_final(seg_attr, atomic_energies.reshape(Z, 1).astype(_f32),
                     seg1, seg2, Wread_0, Winv_0, Wread_1, Winv_1, Wdec)


# speedup vs baseline: 3.0822x; 1.0168x over previous
import functools

import jax
import jax.numpy as jnp
from jax import lax
from jax.experimental import pallas as pl
from jax.experimental.pallas import tpu as pltpu
from jax.experimental.pallas import tpu_sc as plsc

N = 10000
E = 160000
F = 128
Z = 4
G = 16
NB = 8
LM = 4
NE = 3
NP = 4
R_MAX = 5.0
MSG_SCALE = 1.0 / (4.0 * 16.0)

NC = 2
NS = 16
NW = NC * NS
CH = 128
NCHT = E // CH
KPW = (NCHT + NW - 1) // NW
MCH = 64
NMCH = E // MCH
KM = 80
FBN = 4
NPT = N // NS

NBLK = 400
NGRID = N // NBLK
EBLK = 1280
EGRID = E // EBLK

_f32 = jnp.float32
_i32 = jnp.int32


def _sc_vec(pos_t, src, dst):
    mesh = plsc.VectorSubcoreMesh(core_axis_name="c", subcore_axis_name="s")

    @functools.partial(
        pl.kernel,
        out_type=jax.ShapeDtypeStruct((E, 16), _f32),
        mesh=mesh,
        scratch_types=[
            pltpu.VMEM((2, CH), _i32),
            pltpu.VMEM((2, CH), _i32),
            pltpu.VMEM((2, CH, 16), _f32),
            pltpu.VMEM((2, CH, 16), _f32),
            pltpu.SemaphoreType.DMA,
            pltpu.SemaphoreType.DMA,
            pltpu.SemaphoreType.DMA,
            pltpu.SemaphoreType.DMA,
        ],
        compiler_params=pltpu.CompilerParams(use_tc_tiling_on_sc=False),
    )
    def k(pos_hbm, src_hbm, dst_hbm, out_hbm, sv, dv, ps, pd,
          l0, l1, g0, g1):
        wid = lax.axis_index("c") * NS + lax.axis_index("s")
        lsems = (l0, l1)
        gsems = (g0, g1)

        def cbase(c):
            return jnp.minimum(wid + c * NW, NCHT - 1) * CH

        def stage(c, b):
            base = cbase(c)
            pltpu.async_copy(src_hbm.at[pl.ds(base, CH)], sv.at[b], lsems[b])
            pltpu.async_copy(dst_hbm.at[pl.ds(base, CH)], dv.at[b], lsems[b])

        def wait_stage(c, b):
            base = cbase(c)
            pltpu.make_async_copy(src_hbm.at[pl.ds(base, CH)], sv.at[b],
                                  lsems[b]).wait()
            pltpu.make_async_copy(dst_hbm.at[pl.ds(base, CH)], dv.at[b],
                                  lsems[b]).wait()

        def start_gather(b):
            pltpu.async_copy(pos_hbm.at[sv.at[b]], ps.at[b], gsems[b])
            pltpu.async_copy(pos_hbm.at[dv.at[b]], pd.at[b], gsems[b])

        def wait_gather(b):
            pltpu.make_async_copy(pos_hbm.at[sv.at[b]], ps.at[b],
                                  gsems[b]).wait()
            pltpu.make_async_copy(pos_hbm.at[dv.at[b]], pd.at[b],
                                  gsems[b]).wait()

        def halfstep(c, cur, oth, has_next, has_next2):
            @pl.when(has_next)
            def _():
                wait_stage(c + 1, oth)
                start_gather(oth)

            wait_gather(cur)

            def body(i, _):
                ps[cur, i] = pd[cur, i] - ps[cur, i]
                return 0

            lax.fori_loop(0, CH, body, 0, unroll=2)
            pltpu.sync_copy(ps.at[cur], out_hbm.at[pl.ds(cbase(c), CH)])

            @pl.when(has_next2)
            def _():
                stage(c + 2, cur)

        stage(0, 0)
        wait_stage(0, 0)
        start_gather(0)
        stage(1, 1)

        def pair(j, _):
            c0 = 2 * j
            halfstep(c0, 0, 1, c0 + 1 < KPW, c0 + 2 < KPW)
            halfstep(c0 + 1, 1, 0, c0 + 2 < KPW, c0 + 3 < KPW)
            return 0

        lax.fori_loop(0, KPW // 2, pair, 0)

    return k(pos_t, src, dst)


def _tc_edge(vec, shifts, m1_0, m2_0, m3_0, m1_1, m2_1, m3_1):
    def _eye(n):
        return (lax.broadcasted_iota(_i32, (n, n), 0) ==
                lax.broadcasted_iota(_i32, (n, n), 1)).astype(_f32)

    def _t(x, n):
        return lax.dot_general(_eye(n), x, (((1,), (1,)), ((), ())),
                               preferred_element_type=_f32)

    def body(vec_ref, shf_ref, a1, a2, a3, b1, b2, b3, sh_ref, w0_ref, w1_ref):
        vbt = _t(vec_ref[:, 0:3], 3) + _t(shf_ref[...], 3)
        vx = vbt[0:1]
        vy = vbt[1:2]
        vz = vbt[2:3]
        l2 = vx * vx + vy * vy + vz * vz + 1e-12
        length = jnp.sqrt(l2)
        rlen = 1.0 / length
        u = length * (1.0 / R_MAX)
        s3 = 3.0 ** 0.5
        sht = jnp.concatenate(
            [jnp.ones((1, EBLK), _f32), s3 * (vbt * rlen)], axis=0)
        sh_ref[...] = lax.dot_general(sht, _eye(4), (((0,), (0,)), ((), ())),
                                      preferred_element_type=_f32)
        u2 = u * u
        u4 = u2 * u2
        u5 = u4 * u
        poly = 1.0 - 21.0 * u5 + 35.0 * u5 * u - 15.0 * u5 * u2
        cut = jnp.where(u < 1.0, poly, 0.0)
        nvec = (lax.broadcasted_iota(_i32, (NB, 1), 0) + 1).astype(_f32)
        bt = ((2.0 / R_MAX) ** 0.5) * jnp.sin(nvec * (jnp.pi * u)) * rlen
        eft = bt * cut
        for (m1, m2, m3, out_ref) in ((a1, a2, a3, w0_ref),
                                      (b1, b2, b3, w1_ref)):
            x = lax.dot_general(m1[...], eft, (((0,), (0,)), ((), ())),
                                preferred_element_type=_f32)
            x = jax.nn.silu(x)
            x = jax.nn.silu(
                lax.dot_general(m2[...], x, (((0,), (0,)), ((), ())),
                                preferred_element_type=_f32))
            we = lax.dot_general(x, m3[...], (((0,), (0,)), ((), ())),
                                 preferred_element_type=_f32) * MSG_SCALE
            for fb in range(FBN):
                out_ref[fb] = we[:, fb * 32:(fb + 1) * 32]

    wspec = [
        pl.BlockSpec((NB, 64), lambda i: (0, 0)),
        pl.BlockSpec((64, 64), lambda i: (0, 0)),
        pl.BlockSpec((64, F), lambda i: (0, 0)),
    ]
    return pl.pallas_call(
        body,
        grid=(EGRID,),
        in_specs=[
            pl.BlockSpec((EBLK, 16), lambda i: (i, 0)),
            pl.BlockSpec((EBLK, 3), lambda i: (i, 0)),
        ] + wspec + wspec,
        out_specs=[
            pl.BlockSpec((EBLK, 4), lambda i: (i, 0)),
            pl.BlockSpec((FBN, EBLK, 32), lambda i: (0, i, 0)),
            pl.BlockSpec((FBN, EBLK, 32), lambda i: (0, i, 0)),
        ],
        out_shape=[
            jax.ShapeDtypeStruct((E, 4), _f32),
            jax.ShapeDtypeStruct((FBN, E, 32), _f32),
            jax.ShapeDtypeStruct((FBN, E, 32), _f32),
        ],
    )(vec, shifts, m1_0, m2_0, m3_0, m1_1, m2_1, m3_1)


def _tc_embed(node_attrs, batch3, w_embed):
    def body(attr_ref, b3_ref, we_ref, h0_ref, seg_ref):
        attrs = attr_ref[...]
        h0 = jnp.dot(attrs, we_ref[...], preferred_element_type=_f32)
        for fb in range(FBN):
            h0_ref[fb] = h0[:, fb * 32:(fb + 1) * 32]
        bt = b3_ref[0, 0]
        oh = (bt[:, None] ==
              lax.broadcasted_iota(_i32, (NBLK, G), 1)).astype(_f32)
        sa = lax.dot_general(oh, attrs, (((0,), (0,)), ((), ())),
                             preferred_element_type=_f32)

        @pl.when(pl.program_id(0) == 0)
        def _():
            seg_ref[...] = jnp.zeros_like(seg_ref)

        seg_ref[...] += sa

    return pl.pallas_call(
        body,
        grid=(NGRID,),
        in_specs=[
            pl.BlockSpec((NBLK, Z), lambda i: (i, 0)),
            pl.BlockSpec((1, 1, NBLK), lambda i: (i, 0, 0)),
            pl.BlockSpec((Z, F), lambda i: (0, 0)),
        ],
        out_specs=[
            pl.BlockSpec((FBN, NBLK, 32), lambda i: (0, i, 0)),
            pl.BlockSpec((G, Z), lambda i: (0, 0)),
        ],
        out_shape=[
            jax.ShapeDtypeStruct((FBN, N, 32), _f32),
            jax.ShapeDtypeStruct((G, Z), _f32),
        ],
    )(node_attrs, batch3, w_embed)


def _sc_msg(h_flat, sh, we, src, dst, first_layer):
    ht = 32 if first_layer else F
    mch = 80 if first_layer else 64
    nmch = E // mch
    km = -(-nmch // NW)
    km += km % 2
    zfull, zrem = divmod(NPT, mch)
    zcnts = (mch,) * zfull + ((zrem,) if zrem else ())
    mesh = plsc.VectorSubcoreMesh(core_axis_name="c", subcore_axis_name="s")

    @functools.partial(
        pl.kernel,
        out_type=jax.ShapeDtypeStruct((NC, FBN, N, F), _f32),
        mesh=mesh,
        scratch_types=[
            pltpu.VMEM_SHARED((N, F), _f32),
            pltpu.VMEM((2, mch), _i32),
            pltpu.VMEM((4, mch), _i32),
            pltpu.VMEM((2, mch, ht), _f32),
            pltpu.VMEM((2, mch, F), _f32),
            pltpu.VMEM((2, mch, 32), _f32),
            pltpu.VMEM((2, mch * 4 + 16), _f32),
            pltpu.SemaphoreType.DMA,
            pltpu.SemaphoreType.DMA,
            pltpu.SemaphoreType.DMA,
            pltpu.SemaphoreType.DMA,
            pltpu.SemaphoreType.DMA,
            pltpu.SemaphoreType.DMA,
        ],
        compiler_params=pltpu.CompilerParams(use_tc_tiling_on_sc=False),
    )
    def k(h_hbm, sh_hbm, we_hbm, src_hbm, dst_hbm,
          out_hbm, agg, sv, dv, rows, pay, wev, shv,
          ls0, ls1, gs0, gs1, ss0, ss1):
        cc = lax.axis_index("c")
        ss = lax.axis_index("s")
        wid = cc * NS + ss
        r0 = ss * NPT
        lsems = (ls0, ls1)
        gsems = (gs0, gs1)
        ssems = (ss0, ss1)

        def cbase(c):
            return jnp.minimum(wid + c * NW, nmch - 1) * mch

        def is_real(c):
            return wid + c * NW < nmch

        for fb in range(FBN):

            def stage(c, b):
                base = cbase(c)
                pltpu.async_copy(src_hbm.at[pl.ds(base, mch)], sv.at[b],
                                 lsems[b])
                pltpu.async_copy(dst_hbm.at[pl.ds(base, mch)], dv.at[c % 4],
                                 lsems[b])
                pltpu.async_copy(we_hbm.at[fb, pl.ds(base, mch)], wev.at[b],
                                 lsems[b])
                pltpu.async_copy(sh_hbm.at[pl.ds(base * 4, mch * 4)],
                                 shv.at[b, pl.ds(0, mch * 4)], lsems[b])

            def wait_stage(c, b):
                base = cbase(c)
                pltpu.make_async_copy(src_hbm.at[pl.ds(base, mch)], sv.at[b],
                                      lsems[b]).wait()
                pltpu.make_async_copy(dst_hbm.at[pl.ds(base, mch)],
                                      dv.at[c % 4], lsems[b]).wait()
                pltpu.make_async_copy(we_hbm.at[fb, pl.ds(base, mch)],
                                      wev.at[b], lsems[b]).wait()
                pltpu.make_async_copy(sh_hbm.at[pl.ds(base * 4, mch * 4)],
                                      shv.at[b, pl.ds(0, mch * 4)],
                                      lsems[b]).wait()

            def adjust_idx(b):
                if fb > 0:
                    for g4 in range(mch // 16):
                        sv[b, pl.ds(g4 * 16, 16)] = (
                            sv[b, pl.ds(g4 * 16, 16)] + fb * N)

            def start_gather(b):
                pltpu.async_copy(h_hbm.at[sv.at[b]], rows.at[b], gsems[b])

            def wait_gather(b):
                pltpu.make_async_copy(h_hbm.at[sv.at[b]], rows.at[b],
                                      gsems[b]).wait()

            def start_scatter(c, b):
                @pl.when(is_real(c))
                def _():
                    pltpu.async_copy(pay.at[b], agg.at[dv.at[c % 4]],
                                     ssems[b], add=True)

            def wait_scatter(c, b):
                @pl.when(is_real(c))
                def _():
                    pltpu.make_async_copy(pay.at[b], agg.at[dv.at[c % 4]],
                                          ssems[b]).wait()

            def compute(b):
                def ebody(i, _):
                    svec = shv[b, pl.ds(i * 4, 16)]
                    s1 = svec[1]
                    s2 = svec[2]
                    s3 = svec[3]
                    for gg in range(2):
                        o = gg * 16
                        if first_layer:
                            proj = rows[b, i, pl.ds(o, 16)]
                        else:
                            proj = (rows[b, i, pl.ds(o, 16)]
                                    + rows[b, i, pl.ds(32 + o, 16)] * s1
                                    + rows[b, i, pl.ds(64 + o, 16)] * s2
                                    + rows[b, i, pl.ds(96 + o, 16)] * s3)
                        a = proj * wev[b, i, pl.ds(o, 16)]
                        pay[b, i, pl.ds(o, 16)] = a
                        pay[b, i, pl.ds(32 + o, 16)] = a * s1
                        pay[b, i, pl.ds(64 + o, 16)] = a * s2
                        pay[b, i, pl.ds(96 + o, 16)] = a * s3
                    return 0

                lax.fori_loop(0, mch, ebody, 0, unroll=4)

            def zbody(r, _):
                for gg in range(F // 16):
                    pay[0, r, pl.ds(gg * 16, 16)] = jnp.zeros((16,), _f32)
                return 0

            lax.fori_loop(0, mch, zbody, 0)
            off = 0
            for cnt in zcnts:
                pltpu.sync_copy(pay.at[0, pl.ds(0, cnt)],
                                agg.at[pl.ds(r0 + off, cnt)])
                off += cnt
            plsc.subcore_barrier()

            def halfstep(c, cur, oth, has_next, has_next2):
                @pl.when(has_next)
                def _():
                    wait_stage(c + 1, oth)
                    adjust_idx(oth)
                    start_gather(oth)

                wait_gather(cur)

                @pl.when(c >= 2)
                def _():
                    wait_scatter(c - 2, cur)

                compute(cur)
                start_scatter(c, cur)

                @pl.when(has_next2)
                def _():
                    stage(c + 2, cur)

            stage(0, 0)
            wait_stage(0, 0)
            adjust_idx(0)
            start_gather(0)
            stage(1, 1)

            def pair(j, _):
                c0 = 2 * j
                halfstep(c0, 0, 1, c0 + 1 < km, c0 + 2 < km)
                halfstep(c0 + 1, 1, 0, c0 + 2 < km, c0 + 3 < km)
                return 0

            lax.fori_loop(0, km // 2, pair, 0)
            wait_scatter(km - 2, 0)
            wait_scatter(km - 1, 1)
            plsc.subcore_barrier()

            off = 0
            for cnt in (128, 128, 128, 128, NPT - 512):
                for cc_s in range(NC):
                    @pl.when(cc == cc_s)
                    def _():
                        pltpu.sync_copy(
                            agg.at[pl.ds(r0 + off, cnt)],
                            out_hbm.at[cc_s, fb, pl.ds(r0 + off, cnt)])
                off += cnt
            plsc.subcore_barrier()

    return k(h_flat, sh, we, src, dst)


def _tc_node(aggp, h_tab, batch3, wm, ws, first_layer):
    ht = 32 if first_layer else F

    def body(agg_ref, h_ref, b3_ref, wm_ref, ws_ref, hn_ref, seg_ref):
        a = agg_ref[0] + agg_ref[1]
        wmv = wm_ref[...]
        wsv = ws_ref[...]
        ms = []
        bsq = jnp.zeros((NBLK, F), _f32)
        for l in range(LM):
            x = jnp.concatenate(
                [a[fb, :, l * 32:(l + 1) * 32] for fb in range(FBN)], axis=1)
            m = jnp.dot(x, wmv, preferred_element_type=_f32)
            ms.append(m)
            bsq = bsq + m * m
        t = jnp.tanh(bsq)
        inv = None
        for l in range(LM):
            if first_layer:
                if l == 0:
                    hsrc = jnp.concatenate(
                        [h_ref[fb] for fb in range(FBN)], axis=1)
                    hx = jnp.dot(hsrc, wsv, preferred_element_type=_f32)
                else:
                    hx = None
            else:
                hsrc = jnp.concatenate(
                    [h_ref[fb, :, l * 32:(l + 1) * 32] for fb in range(FBN)],
                    axis=1)
                hx = jnp.dot(hsrc, wsv, preferred_element_type=_f32)
            hn = ms[l] + ms[l] * t
            if hx is not None:
                hn = hn + hx
            for fb in range(FBN):
                hn_ref[fb, :, l * 32:(l + 1) * 32] = hn[:, fb * 32:(fb + 1) * 32]
            if l == 0:
                inv = hn
        bt = b3_ref[0, 0]
        oh = (bt[:, None] ==
              lax.broadcasted_iota(_i32, (NBLK, G), 1)).astype(_f32)
        sa = lax.dot_general(oh, inv, (((0,), (0,)), ((), ())),
                             preferred_element_type=_f32)

        @pl.when(pl.program_id(0) == 0)
        def _():
            seg_ref[...] = jnp.zeros_like(seg_ref)

        seg_ref[...] += sa

    hspec = (pl.BlockSpec((FBN, NBLK, 32), lambda i: (0, i, 0)) if first_layer
             else pl.BlockSpec((FBN, NBLK, F), lambda i: (0, i, 0)))
    return pl.pallas_call(
        body,
        grid=(NGRID,),
        in_specs=[
            pl.BlockSpec((NC, FBN, NBLK, F), lambda i: (0, 0, i, 0)),
            hspec,
            pl.BlockSpec((1, 1, NBLK), lambda i: (i, 0, 0)),
            pl.BlockSpec((F, F), lambda i: (0, 0)),
            pl.BlockSpec((F, F), lambda i: (0, 0)),
        ],
        out_specs=[
            pl.BlockSpec((FBN, NBLK, F), lambda i: (0, i, 0)),
            pl.BlockSpec((G, F), lambda i: (0, 0)),
        ],
        out_shape=[
            jax.ShapeDtypeStruct((FBN, N, F), _f32),
            jax.ShapeDtypeStruct((G, F), _f32),
        ],
    )(aggp, h_tab, batch3, wm, ws)


def _tc_final(seg_attr, ae1, seg1, seg2, wr0, wi0, wr1, wi1, wdec):
    def body(sa_ref, ae_ref, s1_ref, s2_ref, wr0_ref, wi0_ref, wr1_ref,
             wi1_ref, wd_ref, out_ref):
        e0 = jnp.dot(sa_ref[...], ae_ref[...], preferred_element_type=_f32)
        s1 = s1_ref[...]
        s2 = s2_ref[...]
        en = (jnp.dot(s1, wr0_ref[...], preferred_element_type=_f32)
              + jnp.dot(s2, wr1_ref[...], preferred_element_type=_f32))
        il = (jnp.dot(s1, wi0_ref[...], preferred_element_type=_f32)
              + jnp.dot(s2, wi1_ref[...], preferred_element_type=_f32))
        out_ref[...] = e0 + en + jnp.dot(il, wd_ref[...],
                                         preferred_element_type=_f32)

    return pl.pallas_call(
        body,
        out_shape=jax.ShapeDtypeStruct((G, NE), _f32),
    )(seg_attr, ae1, seg1, seg2, wr0, wi0, wr1, wi1, wdec)


def kernel(positions, node_attrs, edge_index, shifts, batch, ptr,
           atomic_energies, W_embed, mlp1_0, mlp2_0, mlp3_0, Wmix_0, Wsc_0,
           Wread_0, Winv_0, mlp1_1, mlp2_1, mlp3_1, Wmix_1, Wsc_1, Wread_1,
           Winv_1, Wdec):
    src = edge_index[0].astype(_i32)
    dst = edge_index[1].astype(_i32)
    pos_t = jnp.zeros((N, 16), _f32).at[:, 0:3].set(positions.astype(_f32))
    batch3 = batch.astype(_i32).reshape(NGRID, 1, NBLK)

    vec = _sc_vec(pos_t, src, dst)
    sh, we0, we1 = _tc_edge(vec, shifts.astype(_f32), mlp1_0, mlp2_0, mlp3_0,
                            mlp1_1, mlp2_1, mlp3_1)
    h0_tab, seg_attr = _tc_embed(node_attrs.astype(_f32), batch3, W_embed)

    sh_flat = sh.reshape(E * 4)
    aggp1 = _sc_msg(h0_tab.reshape(FBN * N, 32), sh_flat, we0, src, dst,
                    first_layer=True)
    h1_tab, seg1 = _tc_node(aggp1, h0_tab, batch3, Wmix_0, Wsc_0,
                            first_layer=True)

    aggp2 = _sc_msg(h1_tab.reshape(FBN * N, F), sh_flat, we1, src, dst,
                    first_layer=False)
    _, seg2 = _tc_node(aggp2, h1_tab, batch3, Wmix_1, Wsc_1,
                       first_layer=False)

    return _tc_final(seg_attr, atomic_energies.reshape(Z, 1).astype(_f32),
                     seg1, seg2, Wread_0, Winv_0, Wread_1, Winv_1, Wdec)


# speedup vs baseline: 3.2065x; 1.0403x over previous
import functools

import jax
import jax.numpy as jnp
from jax import lax
from jax.experimental import pallas as pl
from jax.experimental.pallas import tpu as pltpu
from jax.experimental.pallas import tpu_sc as plsc

N = 10000
E = 160000
F = 128
Z = 4
G = 16
NB = 8
LM = 4
NE = 3
NP = 4
R_MAX = 5.0
MSG_SCALE = 1.0 / (4.0 * 16.0)

NC = 2
NS = 16
NW = NC * NS
CH = 128
NCHT = E // CH
KPW = (NCHT + NW - 1) // NW
MCH = 64
NMCH = E // MCH
KM = 80
FBN = 4
NPT = N // NS

NBLK = 400
NGRID = N // NBLK
EBLK = 1280
EGRID = E // EBLK

_f32 = jnp.float32
_i32 = jnp.int32


def _sc_vec(pos_t, src, dst):
    mesh = plsc.VectorSubcoreMesh(core_axis_name="c", subcore_axis_name="s")

    @functools.partial(
        pl.kernel,
        out_type=jax.ShapeDtypeStruct((E, 16), _f32),
        mesh=mesh,
        scratch_types=[
            pltpu.VMEM((2, CH), _i32),
            pltpu.VMEM((2, CH), _i32),
            pltpu.VMEM((2, CH, 16), _f32),
            pltpu.VMEM((2, CH, 16), _f32),
            pltpu.SemaphoreType.DMA,
            pltpu.SemaphoreType.DMA,
            pltpu.SemaphoreType.DMA,
            pltpu.SemaphoreType.DMA,
        ],
        compiler_params=pltpu.CompilerParams(use_tc_tiling_on_sc=False),
    )
    def k(pos_hbm, src_hbm, dst_hbm, out_hbm, sv, dv, ps, pd,
          l0, l1, g0, g1):
        wid = lax.axis_index("c") * NS + lax.axis_index("s")
        lsems = (l0, l1)
        gsems = (g0, g1)

        def cbase(c):
            return jnp.minimum(wid + c * NW, NCHT - 1) * CH

        def stage(c, b):
            base = cbase(c)
            pltpu.async_copy(src_hbm.at[pl.ds(base, CH)], sv.at[b], lsems[b])
            pltpu.async_copy(dst_hbm.at[pl.ds(base, CH)], dv.at[b], lsems[b])

        def wait_stage(c, b):
            base = cbase(c)
            pltpu.make_async_copy(src_hbm.at[pl.ds(base, CH)], sv.at[b],
                                  lsems[b]).wait()
            pltpu.make_async_copy(dst_hbm.at[pl.ds(base, CH)], dv.at[b],
                                  lsems[b]).wait()

        def start_gather(b):
            pltpu.async_copy(pos_hbm.at[sv.at[b]], ps.at[b], gsems[b])
            pltpu.async_copy(pos_hbm.at[dv.at[b]], pd.at[b], gsems[b])

        def wait_gather(b):
            pltpu.make_async_copy(pos_hbm.at[sv.at[b]], ps.at[b],
                                  gsems[b]).wait()
            pltpu.make_async_copy(pos_hbm.at[dv.at[b]], pd.at[b],
                                  gsems[b]).wait()

        def halfstep(c, cur, oth, has_next, has_next2):
            @pl.when(has_next)
            def _():
                wait_stage(c + 1, oth)
                start_gather(oth)

            wait_gather(cur)

            def body(i, _):
                ps[cur, i] = pd[cur, i] - ps[cur, i]
                return 0

            lax.fori_loop(0, CH, body, 0, unroll=2)
            pltpu.sync_copy(ps.at[cur], out_hbm.at[pl.ds(cbase(c), CH)])

            @pl.when(has_next2)
            def _():
                stage(c + 2, cur)

        stage(0, 0)
        wait_stage(0, 0)
        start_gather(0)
        stage(1, 1)

        def pair(j, _):
            c0 = 2 * j
            halfstep(c0, 0, 1, c0 + 1 < KPW, c0 + 2 < KPW)
            halfstep(c0 + 1, 1, 0, c0 + 2 < KPW, c0 + 3 < KPW)
            return 0

        lax.fori_loop(0, KPW // 2, pair, 0)

    return k(pos_t, src, dst)


def _tc_edge_a(vec, shifts, m1, m2, m3):
    def _eye(n):
        return (lax.broadcasted_iota(_i32, (n, n), 0) ==
                lax.broadcasted_iota(_i32, (n, n), 1)).astype(_f32)

    def _t(x, n):
        return lax.dot_general(_eye(n), x, (((1,), (1,)), ((), ())),
                               preferred_element_type=_f32)

    def body(vec_ref, shf_ref, a1, a2, a3, sh_ref, ef_ref, w0_ref):
        vbt = _t(vec_ref[:, 0:3], 3) + _t(shf_ref[...], 3)
        vx = vbt[0:1]
        vy = vbt[1:2]
        vz = vbt[2:3]
        l2 = vx * vx + vy * vy + vz * vz + 1e-12
        length = jnp.sqrt(l2)
        rlen = 1.0 / length
        u = length * (1.0 / R_MAX)
        s3 = 3.0 ** 0.5
        sht = jnp.concatenate(
            [jnp.ones((1, EBLK), _f32), s3 * (vbt * rlen)], axis=0)
        sh_ref[...] = lax.dot_general(sht, _eye(4), (((0,), (0,)), ((), ())),
                                      preferred_element_type=_f32)
        u2 = u * u
        u4 = u2 * u2
        u5 = u4 * u
        poly = 1.0 - 21.0 * u5 + 35.0 * u5 * u - 15.0 * u5 * u2
        cut = jnp.where(u < 1.0, poly, 0.0)
        nvec = (lax.broadcasted_iota(_i32, (NB, 1), 0) + 1).astype(_f32)
        bt = ((2.0 / R_MAX) ** 0.5) * jnp.sin(nvec * (jnp.pi * u)) * rlen
        eft = bt * cut
        ef_ref[...] = eft
        x = lax.dot_general(a1[...], eft, (((0,), (0,)), ((), ())),
                            preferred_element_type=_f32)
        x = jax.nn.silu(x)
        x = jax.nn.silu(
            lax.dot_general(a2[...], x, (((0,), (0,)), ((), ())),
                            preferred_element_type=_f32))
        we = lax.dot_general(x, a3[...], (((0,), (0,)), ((), ())),
                             preferred_element_type=_f32) * MSG_SCALE
        for fb in range(FBN):
            w0_ref[fb] = we[:, fb * 32:(fb + 1) * 32]

    return pl.pallas_call(
        body,
        grid=(EGRID,),
        in_specs=[
            pl.BlockSpec((EBLK, 16), lambda i: (i, 0)),
            pl.BlockSpec((EBLK, 3), lambda i: (i, 0)),
            pl.BlockSpec((NB, 64), lambda i: (0, 0)),
            pl.BlockSpec((64, 64), lambda i: (0, 0)),
            pl.BlockSpec((64, F), lambda i: (0, 0)),
        ],
        out_specs=[
            pl.BlockSpec((EBLK, 4), lambda i: (i, 0)),
            pl.BlockSpec((NB, EBLK), lambda i: (0, i)),
            pl.BlockSpec((FBN, EBLK, 32), lambda i: (0, i, 0)),
        ],
        out_shape=[
            jax.ShapeDtypeStruct((E, 4), _f32),
            jax.ShapeDtypeStruct((NB, E), _f32),
            jax.ShapeDtypeStruct((FBN, E, 32), _f32),
        ],
    )(vec, shifts, m1, m2, m3)


def _tc_edge_b(eft_all, m1, m2, m3):
    def body(ef_ref, b1, b2, b3, w1_ref):
        eft = ef_ref[...]
        x = lax.dot_general(b1[...], eft, (((0,), (0,)), ((), ())),
                            preferred_element_type=_f32)
        x = jax.nn.silu(x)
        x = jax.nn.silu(
            lax.dot_general(b2[...], x, (((0,), (0,)), ((), ())),
                            preferred_element_type=_f32))
        we = lax.dot_general(x, b3[...], (((0,), (0,)), ((), ())),
                             preferred_element_type=_f32) * MSG_SCALE
        for fb in range(FBN):
            w1_ref[fb] = we[:, fb * 32:(fb + 1) * 32]

    return pl.pallas_call(
        body,
        grid=(EGRID,),
        in_specs=[
            pl.BlockSpec((NB, EBLK), lambda i: (0, i)),
            pl.BlockSpec((NB, 64), lambda i: (0, 0)),
            pl.BlockSpec((64, 64), lambda i: (0, 0)),
            pl.BlockSpec((64, F), lambda i: (0, 0)),
        ],
        out_specs=pl.BlockSpec((FBN, EBLK, 32), lambda i: (0, i, 0)),
        out_shape=jax.ShapeDtypeStruct((FBN, E, 32), _f32),
    )(eft_all, m1, m2, m3)


def _tc_embed(node_attrs, batch3, w_embed):
    def body(attr_ref, b3_ref, we_ref, h0_ref, seg_ref):
        attrs = attr_ref[...]
        h0 = jnp.dot(attrs, we_ref[...], preferred_element_type=_f32)
        for fb in range(FBN):
            h0_ref[fb] = h0[:, fb * 32:(fb + 1) * 32]
        bt = b3_ref[0, 0]
        oh = (bt[:, None] ==
              lax.broadcasted_iota(_i32, (NBLK, G), 1)).astype(_f32)
        sa = lax.dot_general(oh, attrs, (((0,), (0,)), ((), ())),
                             preferred_element_type=_f32)

        @pl.when(pl.program_id(0) == 0)
        def _():
            seg_ref[...] = jnp.zeros_like(seg_ref)

        seg_ref[...] += sa

    return pl.pallas_call(
        body,
        grid=(NGRID,),
        in_specs=[
            pl.BlockSpec((NBLK, Z), lambda i: (i, 0)),
            pl.BlockSpec((1, 1, NBLK), lambda i: (i, 0, 0)),
            pl.BlockSpec((Z, F), lambda i: (0, 0)),
        ],
        out_specs=[
            pl.BlockSpec((FBN, NBLK, 32), lambda i: (0, i, 0)),
            pl.BlockSpec((G, Z), lambda i: (0, 0)),
        ],
        out_shape=[
            jax.ShapeDtypeStruct((FBN, N, 32), _f32),
            jax.ShapeDtypeStruct((G, Z), _f32),
        ],
    )(node_attrs, batch3, w_embed)


def _sc_msg(h_flat, sh, we, src, dst, first_layer):
    ht = 32 if first_layer else F
    mch = 80 if first_layer else 64
    nmch = E // mch
    km = -(-nmch // NW)
    km += km % 2
    zfull, zrem = divmod(NPT, mch)
    zcnts = (mch,) * zfull + ((zrem,) if zrem else ())
    mesh = plsc.VectorSubcoreMesh(core_axis_name="c", subcore_axis_name="s")

    @functools.partial(
        pl.kernel,
        out_type=jax.ShapeDtypeStruct((NC, FBN, N, F), _f32),
        mesh=mesh,
        scratch_types=[
            pltpu.VMEM_SHARED((N, F), _f32),
            pltpu.VMEM((2, mch), _i32),
            pltpu.VMEM((4, mch), _i32),
            pltpu.VMEM((2, mch, ht), _f32),
            pltpu.VMEM((2, mch, F), _f32),
            pltpu.VMEM((2, mch, 32), _f32),
            pltpu.VMEM((2, mch * 4 + 16), _f32),
            pltpu.SemaphoreType.DMA,
            pltpu.SemaphoreType.DMA,
            pltpu.SemaphoreType.DMA,
            pltpu.SemaphoreType.DMA,
            pltpu.SemaphoreType.DMA,
            pltpu.SemaphoreType.DMA,
        ],
        compiler_params=pltpu.CompilerParams(use_tc_tiling_on_sc=False),
    )
    def k(h_hbm, sh_hbm, we_hbm, src_hbm, dst_hbm,
          out_hbm, agg, sv, dv, rows, pay, wev, shv,
          ls0, ls1, gs0, gs1, ss0, ss1):
        cc = lax.axis_index("c")
        ss = lax.axis_index("s")
        wid = cc * NS + ss
        r0 = ss * NPT
        lsems = (ls0, ls1)
        gsems = (gs0, gs1)
        ssems = (ss0, ss1)

        def cbase(c):
            return jnp.minimum(wid + c * NW, nmch - 1) * mch

        def is_real(c):
            return wid + c * NW < nmch

        for fb in range(FBN):

            def stage(c, b):
                base = cbase(c)
                pltpu.async_copy(src_hbm.at[pl.ds(base, mch)], sv.at[b],
                                 lsems[b])
                pltpu.async_copy(dst_hbm.at[pl.ds(base, mch)], dv.at[c % 4],
                                 lsems[b])
                pltpu.async_copy(we_hbm.at[fb, pl.ds(base, mch)], wev.at[b],
                                 lsems[b])
                pltpu.async_copy(sh_hbm.at[pl.ds(base * 4, mch * 4)],
                                 shv.at[b, pl.ds(0, mch * 4)], lsems[b])

            def wait_stage(c, b):
                base = cbase(c)
                pltpu.make_async_copy(src_hbm.at[pl.ds(base, mch)], sv.at[b],
                                      lsems[b]).wait()
                pltpu.make_async_copy(dst_hbm.at[pl.ds(base, mch)],
                                      dv.at[c % 4], lsems[b]).wait()
                pltpu.make_async_copy(we_hbm.at[fb, pl.ds(base, mch)],
                                      wev.at[b], lsems[b]).wait()
                pltpu.make_async_copy(sh_hbm.at[pl.ds(base * 4, mch * 4)],
                                      shv.at[b, pl.ds(0, mch * 4)],
                                      lsems[b]).wait()

            def adjust_idx(b):
                if fb > 0:
                    for g4 in range(mch // 16):
                        sv[b, pl.ds(g4 * 16, 16)] = (
                            sv[b, pl.ds(g4 * 16, 16)] + fb * N)

            def start_gather(b):
                pltpu.async_copy(h_hbm.at[sv.at[b]], rows.at[b], gsems[b])

            def wait_gather(b):
                pltpu.make_async_copy(h_hbm.at[sv.at[b]], rows.at[b],
                                      gsems[b]).wait()

            def start_scatter(c, b):
                @pl.when(is_real(c))
                def _():
                    pltpu.async_copy(pay.at[b], agg.at[dv.at[c % 4]],
                                     ssems[b], add=True)

            def wait_scatter(c, b):
                @pl.when(is_real(c))
                def _():
                    pltpu.make_async_copy(pay.at[b], agg.at[dv.at[c % 4]],
                                          ssems[b]).wait()

            def compute(b):
                def ebody(i, _):
                    svec = shv[b, pl.ds(i * 4, 16)]
                    s1 = svec[1]
                    s2 = svec[2]
                    s3 = svec[3]
                    for gg in range(2):
                        o = gg * 16
                        if first_layer:
                            proj = rows[b, i, pl.ds(o, 16)]
                        else:
                            proj = (rows[b, i, pl.ds(o, 16)]
                                    + rows[b, i, pl.ds(32 + o, 16)] * s1
                                    + rows[b, i, pl.ds(64 + o, 16)] * s2
                                    + rows[b, i, pl.ds(96 + o, 16)] * s3)
                        a = proj * wev[b, i, pl.ds(o, 16)]
                        pay[b, i, pl.ds(o, 16)] = a
                        pay[b, i, pl.ds(32 + o, 16)] = a * s1
                        pay[b, i, pl.ds(64 + o, 16)] = a * s2
                        pay[b, i, pl.ds(96 + o, 16)] = a * s3
                    return 0

                lax.fori_loop(0, mch, ebody, 0, unroll=4)

            def zbody(r, _):
                for gg in range(F // 16):
                    pay[0, r, pl.ds(gg * 16, 16)] = jnp.zeros((16,), _f32)
                return 0

            lax.fori_loop(0, mch, zbody, 0)
            off = 0
            for cnt in zcnts:
                pltpu.sync_copy(pay.at[0, pl.ds(0, cnt)],
                                agg.at[pl.ds(r0 + off, cnt)])
                off += cnt
            plsc.subcore_barrier()

            def halfstep(c, cur, oth, has_next, has_next2):
                @pl.when(has_next)
                def _():
                    wait_stage(c + 1, oth)
                    adjust_idx(oth)
                    start_gather(oth)

                wait_gather(cur)

                @pl.when(c >= 2)
                def _():
                    wait_scatter(c - 2, cur)

                compute(cur)
                start_scatter(c, cur)

                @pl.when(has_next2)
                def _():
                    stage(c + 2, cur)

            stage(0, 0)
            wait_stage(0, 0)
            adjust_idx(0)
            start_gather(0)
            stage(1, 1)

            def pair(j, _):
                c0 = 2 * j
                halfstep(c0, 0, 1, c0 + 1 < km, c0 + 2 < km)
                halfstep(c0 + 1, 1, 0, c0 + 2 < km, c0 + 3 < km)
                return 0

            lax.fori_loop(0, km // 2, pair, 0)
            wait_scatter(km - 2, 0)
            wait_scatter(km - 1, 1)
            plsc.subcore_barrier()

            off = 0
            for cnt in (128, 128, 128, 128, NPT - 512):
                for cc_s in range(NC):
                    @pl.when(cc == cc_s)
                    def _():
                        pltpu.sync_copy(
                            agg.at[pl.ds(r0 + off, cnt)],
                            out_hbm.at[cc_s, fb, pl.ds(r0 + off, cnt)])
                off += cnt
            plsc.subcore_barrier()

    return k(h_flat, sh, we, src, dst)


def _tc_node(aggp, h_tab, batch3, wm, ws, first_layer, emit_h=True):
    ht = 32 if first_layer else F

    def body(agg_ref, h_ref, b3_ref, wm_ref, ws_ref, *out_refs):
        if emit_h:
            hn_ref, seg_ref = out_refs
        else:
            (seg_ref,) = out_refs
        a = agg_ref[0] + agg_ref[1]
        wmv = wm_ref[...]
        wsv = ws_ref[...]
        ms = []
        bsq = jnp.zeros((NBLK, F), _f32)
        for l in range(LM):
            x = jnp.concatenate(
                [a[fb, :, l * 32:(l + 1) * 32] for fb in range(FBN)], axis=1)
            m = jnp.dot(x, wmv, preferred_element_type=_f32)
            ms.append(m)
            bsq = bsq + m * m
        t = jnp.tanh(bsq)
        inv = None
        for l in range(LM):
            if first_layer:
                if l == 0:
                    hsrc = jnp.concatenate(
                        [h_ref[fb] for fb in range(FBN)], axis=1)
                    hx = jnp.dot(hsrc, wsv, preferred_element_type=_f32)
                else:
                    hx = None
            else:
                hsrc = jnp.concatenate(
                    [h_ref[fb, :, l * 32:(l + 1) * 32] for fb in range(FBN)],
                    axis=1)
                hx = jnp.dot(hsrc, wsv, preferred_element_type=_f32)
            hn = ms[l] + ms[l] * t
            if hx is not None:
                hn = hn + hx
            if emit_h:
                for fb in range(FBN):
                    hn_ref[fb, :, l * 32:(l + 1) * 32] = \
                        hn[:, fb * 32:(fb + 1) * 32]
            if l == 0:
                inv = hn
            if not emit_h and l > 0:
                continue
        bt = b3_ref[0, 0]
        oh = (bt[:, None] ==
              lax.broadcasted_iota(_i32, (NBLK, G), 1)).astype(_f32)
        sa = lax.dot_general(oh, inv, (((0,), (0,)), ((), ())),
                             preferred_element_type=_f32)

        @pl.when(pl.program_id(0) == 0)
        def _():
            seg_ref[...] = jnp.zeros_like(seg_ref)

        seg_ref[...] += sa

    hspec = (pl.BlockSpec((FBN, NBLK, 32), lambda i: (0, i, 0)) if first_layer
             else pl.BlockSpec((FBN, NBLK, F), lambda i: (0, i, 0)))
    return pl.pallas_call(
        body,
        grid=(NGRID,),
        in_specs=[
            pl.BlockSpec((NC, FBN, NBLK, F), lambda i: (0, 0, i, 0)),
            hspec,
            pl.BlockSpec((1, 1, NBLK), lambda i: (i, 0, 0)),
            pl.BlockSpec((F, F), lambda i: (0, 0)),
            pl.BlockSpec((F, F), lambda i: (0, 0)),
        ],
        out_specs=([pl.BlockSpec((FBN, NBLK, F), lambda i: (0, i, 0))]
                   if emit_h else [])
        + [pl.BlockSpec((G, F), lambda i: (0, 0))],
        out_shape=([jax.ShapeDtypeStruct((FBN, N, F), _f32)]
                   if emit_h else [])
        + [jax.ShapeDtypeStruct((G, F), _f32)],
    )(aggp, h_tab, batch3, wm, ws)


def _tc_final(seg_attr, ae1, seg1, seg2, wr0, wi0, wr1, wi1, wdec):
    def body(sa_ref, ae_ref, s1_ref, s2_ref, wr0_ref, wi0_ref, wr1_ref,
             wi1_ref, wd_ref, out_ref):
        e0 = jnp.dot(sa_ref[...], ae_ref[...], preferred_element_type=_f32)
        s1 = s1_ref[...]
        s2 = s2_ref[...]
        en = (jnp.dot(s1, wr0_ref[...], preferred_element_type=_f32)
              + jnp.dot(s2, wr1_ref[...], preferred_element_type=_f32))
        il = (jnp.dot(s1, wi0_ref[...], preferred_element_type=_f32)
              + jnp.dot(s2, wi1_ref[...], preferred_element_type=_f32))
        out_ref[...] = e0 + en + jnp.dot(il, wd_ref[...],
                                         preferred_element_type=_f32)

    return pl.pallas_call(
        body,
        out_shape=jax.ShapeDtypeStruct((G, NE), _f32),
    )(seg_attr, ae1, seg1, seg2, wr0, wi0, wr1, wi1, wdec)


def kernel(positions, node_attrs, edge_index, shifts, batch, ptr,
           atomic_energies, W_embed, mlp1_0, mlp2_0, mlp3_0, Wmix_0, Wsc_0,
           Wread_0, Winv_0, mlp1_1, mlp2_1, mlp3_1, Wmix_1, Wsc_1, Wread_1,
           Winv_1, Wdec):
    src = edge_index[0].astype(_i32)
    dst = edge_index[1].astype(_i32)
    pos_t = jnp.zeros((N, 16), _f32).at[:, 0:3].set(positions.astype(_f32))
    batch3 = batch.astype(_i32).reshape(NGRID, 1, NBLK)

    vec = _sc_vec(pos_t, src, dst)
    sh, eft_all, we0 = _tc_edge_a(vec, shifts.astype(_f32), mlp1_0, mlp2_0,
                                  mlp3_0)
    we1 = _tc_edge_b(eft_all, mlp1_1, mlp2_1, mlp3_1)
    h0_tab, seg_attr = _tc_embed(node_attrs.astype(_f32), batch3, W_embed)

    sh_flat = sh.reshape(E * 4)
    aggp1 = _sc_msg(h0_tab.reshape(FBN * N, 32), sh_flat, we0, src, dst,
                    first_layer=True)
    h1_tab, seg1 = _tc_node(aggp1, h0_tab, batch3, Wmix_0, Wsc_0,
                            first_layer=True)

    aggp2 = _sc_msg(h1_tab.reshape(FBN * N, F), sh_flat, we1, src, dst,
                    first_layer=False)
    seg2 = _tc_node(aggp2, h1_tab, batch3, Wmix_1, Wsc_1,
                    first_layer=False, emit_h=False)[0]

    return _tc_final(seg_attr, atomic_energies.reshape(Z, 1).astype(_f32),
                     seg1, seg2, Wread_0, Winv_0, Wread_1, Winv_1, Wdec)
